# Initial kernel scaffold; baseline (speedup 1.0000x reference)
import jax
import jax.numpy as jnp
from jax.experimental import pallas as pl


def kernel(x, adj, key, W, b, ec_W0, ec_b0, ec_W1, ec_b1, ec_W2, ec_b2, ag_W0, ag_b0, ag_W1, ag_b1, ag_W2, ag_b2):
    raise NotImplementedError("write your pallas kernel here")


# speedup vs baseline: 2.8934x; 2.8934x over previous
import functools

import jax
import jax.numpy as jnp
from jax import lax
from jax.experimental import pallas as pl
from jax.experimental.pallas import tpu as pltpu
from jax.experimental.pallas import tpu_sc as plsc

F = 128
MIN_NORM = 1e-15

NC = 2
NS = 16
NW = NC * NS


def _rnorm(x):
    return jnp.clip(jnp.sqrt(jnp.sum(x * x, axis=-1, keepdims=True)), MIN_NORM, None)


def _artanh(z):
    z = jnp.clip(z, -1.0 + 1e-7, 1.0 - 1e-7)
    return 0.5 * jnp.log((1.0 + z) / (1.0 - z))


def _proj(x):
    maxnorm = 1.0 - 4e-3
    n = _rnorm(x)
    return jnp.where(n > maxnorm, x / n * maxnorm, x)


def _expmap0(u):
    un = _rnorm(u)
    return jnp.tanh(un) * u / un


def _logmap0(p):
    pn = _rnorm(p)
    return _artanh(pn) * p / pn


def _mm(a, b_t):
    return lax.dot_general(a, b_t, (((1,), (1,)), ((), ())),
                           preferred_element_type=jnp.float32)


def _node_prep_body(x_ref, w_ref, hb_ref, ht_ref):
    x = x_ref[...]
    w = w_ref[...]
    hb = hb_ref[...]
    xn = _rnorm(x)
    mx = _mm(x, w)
    mxn = _rnorm(mx)
    res = jnp.tanh(mxn / xn * _artanh(xn)) * mx / mxn
    res = _proj(res)
    x2 = jnp.sum(res * res, -1, keepdims=True)
    y2 = jnp.sum(hb * hb, -1, keepdims=True)
    xy = jnp.sum(res * hb, -1, keepdims=True)
    num = (1.0 + 2.0 * xy + y2) * res + (1.0 - x2) * hb
    den = 1.0 + 2.0 * xy + x2 * y2
    h = _proj(num / jnp.clip(den, MIN_NORM, None))
    ht_ref[...] = _logmap0(h)


def _node_prep(x, w, hyp_bias):
    n = x.shape[0]
    bn = 1000
    return pl.pallas_call(
        _node_prep_body,
        grid=(n // bn,),
        in_specs=[
            pl.BlockSpec((bn, F), lambda i: (i, 0)),
            pl.BlockSpec((F, F), lambda i: (0, 0)),
            pl.BlockSpec((1, F), lambda i: (0, 0)),
        ],
        out_specs=pl.BlockSpec((bn, F), lambda i: (i, 0)),
        out_shape=jax.ShapeDtypeStruct((n, F), jnp.float32),
    )(x, w, hyp_bias)


def _edge_mlp_body(gr_ref, gs_ref, ar_ref, as_ref, b0_ref, w1_ref, b1_ref,
                   w2_ref, b2_ref, xs_ref, xsq_ref):
    gr = gr_ref[...]
    gs = gs_ref[...]
    h1 = jax.nn.gelu(_mm(gr, ar_ref[...]) + _mm(gs, as_ref[...]) + b0_ref[...])
    h2 = jax.nn.gelu(_mm(h1, w1_ref[...]) + b1_ref[...])
    xs = _mm(h2, w2_ref[...]) + b2_ref[...]
    xs_ref[...] = xs
    xsq_ref[...] = xs * xs


def _edge_mlp(gr, gs, a_r, a_s, b0, w1, b1, w2, b2):
    e = gr.shape[0]
    be = 3200
    h = a_r.shape[0]
    full = lambda shp: pl.BlockSpec(shp, lambda i: (0, 0))
    return pl.pallas_call(
        _edge_mlp_body,
        grid=(e // be,),
        in_specs=[
            pl.BlockSpec((be, F), lambda i: (i, 0)),
            pl.BlockSpec((be, F), lambda i: (i, 0)),
            full((h, F)), full((h, F)), full((1, h)),
            full((h, h)), full((1, h)),
            full((F, h)), full((1, F)),
        ],
        out_specs=[
            pl.BlockSpec((be, F), lambda i: (i, 0)),
            pl.BlockSpec((be, F), lambda i: (i, 0)),
        ],
        out_shape=[
            jax.ShapeDtypeStruct((e, F), jnp.float32),
            jax.ShapeDtypeStruct((e, F), jnp.float32),
        ],
    )(gr, gs, a_r, a_s, b0, w1, b1, w2, b2)


def _stats_body(sp_ref, qp_ref, cp_ref, xm_ref, xv_ref, bnd_ref):
    s = sp_ref[0] + sp_ref[1]
    q = qp_ref[0] + qp_ref[1]
    cnt = (cp_ref[0] + cp_ref[1])[:, :1]
    cc = jnp.clip(cnt, 1.0, None)
    xm = s / cc
    xv = jnp.maximum(q / cc - xm * xm, 0.0)
    xm_ref[...] = xm
    xv_ref[...] = xv
    bnd_ref[...] = xm + jnp.sqrt(xv * jnp.clip(cnt - 1.0, 0.0, None))


def _stats(sp, qp, cp):
    n = sp.shape[1]
    bn = 1000
    o = pl.BlockSpec((bn, F), lambda i: (i, 0))
    return pl.pallas_call(
        _stats_body,
        grid=(n // bn,),
        in_specs=[
            pl.BlockSpec((2, bn, F), lambda i: (0, i, 0)),
            pl.BlockSpec((2, bn, F), lambda i: (0, i, 0)),
            pl.BlockSpec((2, bn, 16), lambda i: (0, i, 0)),
        ],
        out_specs=[o, o, o],
        out_shape=[jax.ShapeDtypeStruct((n, F), jnp.float32)] * 3,
    )(sp, qp, cp)


def _expw_body(xs_ref, bd_ref, e1_ref, n1_ref, e10_ref, n10_ref):
    xs = xs_ref[...]
    u = xs - bd_ref[...]
    e1 = jnp.exp(u)
    e10 = jnp.exp(10.0 * u)
    e1_ref[...] = e1
    n1_ref[...] = xs * e1
    e10_ref[...] = e10
    n10_ref[...] = xs * e10


def _expw(xs, bndr):
    e = xs.shape[0]
    be = 3200
    sp = pl.BlockSpec((be, F), lambda i: (i, 0))
    return pl.pallas_call(
        _expw_body,
        grid=(e // be,),
        in_specs=[sp, sp],
        out_specs=[sp, sp, sp, sp],
        out_shape=[jax.ShapeDtypeStruct((e, F), jnp.float32)] * 4,
    )(xs, bndr)


def _final_body(xm_ref, xv_ref, d1_ref, n1_ref, d10_ref, n10_ref,
                w0_ref, b0_ref, w1_ref, b1_ref, w2_ref, b2_ref, out_ref):
    xm = xm_ref[...]
    xv = xv_ref[...]
    sm1 = (n1_ref[0] + n1_ref[1]) / jnp.clip(d1_ref[0] + d1_ref[1], MIN_NORM, None)
    sm10 = (n10_ref[0] + n10_ref[1]) / jnp.clip(d10_ref[0] + d10_ref[1], MIN_NORM, None)
    cat = jnp.concatenate([xm, xv, sm1, sm10], axis=-1)
    h = jax.nn.gelu(_mm(cat, w0_ref[...]) + b0_ref[...])
    h = jax.nn.gelu(_mm(h, w1_ref[...]) + b1_ref[...])
    xa = _mm(h, w2_ref[...]) + b2_ref[...]
    xa = _proj(_expmap0(xa))
    xt = jax.nn.relu(_logmap0(xa))
    out_ref[...] = _proj(_expmap0(xt))


def _final(xm, xv, d1p, n1p, d10p, n10p, w0, b0, w1, b1, w2, b2):
    n = xm.shape[0]
    bn = 1000
    h = w0.shape[0]
    nb = pl.BlockSpec((bn, F), lambda i: (i, 0))
    pb = pl.BlockSpec((2, bn, F), lambda i: (0, i, 0))
    full = lambda shp: pl.BlockSpec(shp, lambda i: (0, 0))
    return pl.pallas_call(
        _final_body,
        grid=(n // bn,),
        in_specs=[nb, nb, pb, pb, pb, pb,
                  full((h, 4 * F)), full((1, h)),
                  full((h, h)), full((1, h)),
                  full((F, h)), full((1, F))],
        out_specs=nb,
        out_shape=jax.ShapeDtypeStruct((n, F), jnp.float32),
    )(xm, xv, d1p, n1p, d10p, n10p, w0, b0, w1, b1, w2, b2)


_CHUNK = 80


def _sc_mesh():
    return plsc.VectorSubcoreMesh(core_axis_name="c", subcore_axis_name="s",
                                  num_cores=NC, num_subcores=NS)


def _sc_gather(table, idx):
    e = idx.shape[0]
    per_w = e // NW
    iters = per_w // _CHUNK
    d = table.shape[1]

    @functools.partial(
        pl.kernel,
        mesh=_sc_mesh(),
        out_type=jax.ShapeDtypeStruct((e, d), jnp.float32),
        scratch_types=[
            pltpu.VMEM((_CHUNK,), jnp.int32),
            pltpu.VMEM((_CHUNK, d), jnp.float32),
            pltpu.SemaphoreType.DMA,
        ],
    )
    def k(table_hbm, idx_hbm, out_hbm, idx_v, rows_v, sem):
        c = lax.axis_index("c")
        s = lax.axis_index("s")
        base = (s * NC + c) * per_w

        def body(i, carry):
            off = base + i * _CHUNK
            pltpu.sync_copy(idx_hbm.at[pl.ds(off, _CHUNK)], idx_v)
            pltpu.async_copy(table_hbm.at[idx_v], rows_v, sem).wait()
            pltpu.sync_copy(rows_v, out_hbm.at[pl.ds(off, _CHUNK)])
            return carry

        lax.fori_loop(0, iters, body, 0)

    return k(table, idx)


def _sc_scatter_add(vals, idx, zeros, n):
    e, d = vals.shape
    per_w = e // NW
    iters = per_w // _CHUNK
    rows_t = (n // NS) // 8 * 8
    tail = n - NS * rows_t

    @functools.partial(
        pl.kernel,
        mesh=_sc_mesh(),
        out_type=jax.ShapeDtypeStruct((NC, n, d), jnp.float32),
        scratch_types=[
            pltpu.VMEM((_CHUNK,), jnp.int32),
            pltpu.VMEM((_CHUNK, d), jnp.float32),
            pltpu.VMEM_SHARED((n, d), jnp.float32),
            pltpu.SemaphoreType.DMA,
        ],
    )
    def k(vals_hbm, idx_hbm, zeros_hbm, out_hbm, idx_v, vals_v, acc_sh, sem):
        c = lax.axis_index("c")
        s = lax.axis_index("s")
        pltpu.sync_copy(zeros_hbm.at[pl.ds(s * rows_t, rows_t)],
                        acc_sh.at[pl.ds(s * rows_t, rows_t)])
        if tail:
            @pl.when(s == 0)
            def _():
                pltpu.sync_copy(zeros_hbm.at[pl.ds(NS * rows_t, tail)],
                                acc_sh.at[pl.ds(NS * rows_t, tail)])
        plsc.subcore_barrier()
        base = (s * NC + c) * per_w

        def body(i, carry):
            off = base + i * _CHUNK
            pltpu.sync_copy(idx_hbm.at[pl.ds(off, _CHUNK)], idx_v)
            pltpu.sync_copy(vals_hbm.at[pl.ds(off, _CHUNK)], vals_v)
            pltpu.sync_copy(vals_v, acc_sh.at[idx_v], add=True)
            return carry

        lax.fori_loop(0, iters, body, 0)
        plsc.subcore_barrier()
        pltpu.sync_copy(acc_sh.at[pl.ds(s * rows_t, rows_t)],
                        out_hbm.at[c, pl.ds(s * rows_t, rows_t)])
        if tail:
            @pl.when(s == 0)
            def _():
                pltpu.sync_copy(acc_sh.at[pl.ds(NS * rows_t, tail)],
                                out_hbm.at[c, pl.ds(NS * rows_t, tail)])

    return k(vals, idx, zeros)


def kernel(x, adj, key, W, b, ec_W0, ec_b0, ec_W1, ec_b1, ec_W2, ec_b2,
           ag_W0, ag_b0, ag_W1, ag_b1, ag_W2, ag_b2):
    n = x.shape[0]
    s, r = adj[0], adj[1]

    bias = b.reshape(1, -1)
    bn = jnp.clip(jnp.sqrt(jnp.sum(bias * bias, -1, keepdims=True)), MIN_NORM, None)
    eb = jnp.tanh(bn) * bias / bn
    ebn = jnp.clip(jnp.sqrt(jnp.sum(eb * eb, -1, keepdims=True)), MIN_NORM, None)
    maxnorm = 1.0 - 4e-3
    hyp_bias = jnp.where(ebn > maxnorm, eb / ebn * maxnorm, eb)
    a_r = ec_W0[:, :F] - ec_W0[:, 2 * F:]
    a_s = ec_W0[:, F:2 * F] + ec_W0[:, 2 * F:]
    row = lambda v: v.reshape(1, -1)

    ht = _node_prep(x, W, hyp_bias)
    ghts = _sc_gather(ht, s)
    ghtr = _sc_gather(ht, r)
    xs, xsq = _edge_mlp(ghtr, ghts, a_r, a_s, row(ec_b0), ec_W1, row(ec_b1),
                        ec_W2, row(ec_b2))

    z128 = jnp.zeros((n, F), jnp.float32)
    z16 = jnp.zeros((n, 16), jnp.float32)
    ones16 = jnp.ones((adj.shape[1], 16), jnp.float32)
    sp = _sc_scatter_add(xs, r, z128, n)
    qp = _sc_scatter_add(xsq, r, z128, n)
    cp = _sc_scatter_add(ones16, r, z16, n)
    xm, xv, bnd = _stats(sp, qp, cp)

    bndr = _sc_gather(bnd, r)
    e1, n1, e10, n10 = _expw(xs, bndr)
    d1p = _sc_scatter_add(e1, r, z128, n)
    n1p = _sc_scatter_add(n1, r, z128, n)
    d10p = _sc_scatter_add(e10, r, z128, n)
    n10p = _sc_scatter_add(n10, r, z128, n)

    out = _final(xm, xv, d1p, n1p, d10p, n10p,
                 ag_W0, row(ag_b0), ag_W1, row(ag_b1), ag_W2, row(ag_b2))
    return (out, adj)


# speedup vs baseline: 3.7057x; 1.2807x over previous
import functools

import jax
import jax.numpy as jnp
from jax import lax
from jax.experimental import pallas as pl
from jax.experimental.pallas import tpu as pltpu
from jax.experimental.pallas import tpu_sc as plsc

F = 128
MIN_NORM = 1e-15

NC = 2
NS = 16
NW = NC * NS


def _rnorm(x):
    return jnp.clip(jnp.sqrt(jnp.sum(x * x, axis=-1, keepdims=True)), MIN_NORM, None)


def _artanh(z):
    z = jnp.clip(z, -1.0 + 1e-7, 1.0 - 1e-7)
    return 0.5 * jnp.log((1.0 + z) / (1.0 - z))


def _proj(x):
    maxnorm = 1.0 - 4e-3
    n = _rnorm(x)
    return jnp.where(n > maxnorm, x / n * maxnorm, x)


def _expmap0(u):
    un = _rnorm(u)
    return jnp.tanh(un) * u / un


def _logmap0(p):
    pn = _rnorm(p)
    return _artanh(pn) * p / pn


def _mm(a, b_t):
    return lax.dot_general(a, b_t, (((1,), (1,)), ((), ())),
                           preferred_element_type=jnp.float32)


def _node_prep_body(x_ref, w_ref, hb_ref, ht_ref):
    x = x_ref[...]
    w = w_ref[...]
    hb = hb_ref[...]
    xn = _rnorm(x)
    mx = _mm(x, w)
    mxn = _rnorm(mx)
    res = jnp.tanh(mxn / xn * _artanh(xn)) * mx / mxn
    res = _proj(res)
    x2 = jnp.sum(res * res, -1, keepdims=True)
    y2 = jnp.sum(hb * hb, -1, keepdims=True)
    xy = jnp.sum(res * hb, -1, keepdims=True)
    num = (1.0 + 2.0 * xy + y2) * res + (1.0 - x2) * hb
    den = 1.0 + 2.0 * xy + x2 * y2
    h = _proj(num / jnp.clip(den, MIN_NORM, None))
    ht_ref[...] = _logmap0(h)


def _node_prep(x, w, hyp_bias):
    n = x.shape[0]
    bn = 1000
    return pl.pallas_call(
        _node_prep_body,
        grid=(n // bn,),
        in_specs=[
            pl.BlockSpec((bn, F), lambda i: (i, 0)),
            pl.BlockSpec((F, F), lambda i: (0, 0)),
            pl.BlockSpec((1, F), lambda i: (0, 0)),
        ],
        out_specs=pl.BlockSpec((bn, F), lambda i: (i, 0)),
        out_shape=jax.ShapeDtypeStruct((n, F), jnp.float32),
    )(x, w, hyp_bias)


def _edge_mlp_body(gr_ref, gs_ref, ar_ref, as_ref, b0_ref, w1_ref, b1_ref,
                   w2_ref, b2_ref, xs_ref, xsq_ref):
    gr = gr_ref[...]
    gs = gs_ref[...]
    h1 = jax.nn.gelu(_mm(gr, ar_ref[...]) + _mm(gs, as_ref[...]) + b0_ref[...])
    h2 = jax.nn.gelu(_mm(h1, w1_ref[...]) + b1_ref[...])
    xs = _mm(h2, w2_ref[...]) + b2_ref[...]
    xs_ref[...] = xs
    xsq_ref[...] = xs * xs


def _edge_mlp(gr, gs, a_r, a_s, b0, w1, b1, w2, b2):
    e = gr.shape[0]
    be = 3200
    h = a_r.shape[0]
    full = lambda shp: pl.BlockSpec(shp, lambda i: (0, 0))
    return pl.pallas_call(
        _edge_mlp_body,
        grid=(e // be,),
        in_specs=[
            pl.BlockSpec((be, F), lambda i: (i, 0)),
            pl.BlockSpec((be, F), lambda i: (i, 0)),
            full((h, F)), full((h, F)), full((1, h)),
            full((h, h)), full((1, h)),
            full((F, h)), full((1, F)),
        ],
        out_specs=[
            pl.BlockSpec((be, F), lambda i: (i, 0)),
            pl.BlockSpec((be, F), lambda i: (i, 0)),
        ],
        out_shape=[
            jax.ShapeDtypeStruct((e, F), jnp.float32),
            jax.ShapeDtypeStruct((e, F), jnp.float32),
        ],
    )(gr, gs, a_r, a_s, b0, w1, b1, w2, b2)


def _stats_body(sp_ref, qp_ref, cp_ref, xm_ref, xv_ref, bnd_ref):
    s = sp_ref[0] + sp_ref[1]
    q = qp_ref[0] + qp_ref[1]
    cnt = (cp_ref[0] + cp_ref[1])[:, :1]
    cc = jnp.clip(cnt, 1.0, None)
    xm = s / cc
    xv = jnp.maximum(q / cc - xm * xm, 0.0)
    xm_ref[...] = xm
    xv_ref[...] = xv
    bnd_ref[...] = xm + jnp.sqrt(xv * jnp.clip(cnt - 1.0, 0.0, None))


def _stats(sp, qp, cp):
    n = sp.shape[1]
    bn = 1000
    o = pl.BlockSpec((bn, F), lambda i: (i, 0))
    return pl.pallas_call(
        _stats_body,
        grid=(n // bn,),
        in_specs=[
            pl.BlockSpec((2, bn, F), lambda i: (0, i, 0)),
            pl.BlockSpec((2, bn, F), lambda i: (0, i, 0)),
            pl.BlockSpec((2, bn, 16), lambda i: (0, i, 0)),
        ],
        out_specs=[o, o, o],
        out_shape=[jax.ShapeDtypeStruct((n, F), jnp.float32)] * 3,
    )(sp, qp, cp)


def _expw_body(xs_ref, bd_ref, e1_ref, n1_ref, e10_ref, n10_ref):
    xs = xs_ref[...]
    u = xs - bd_ref[...]
    e1 = jnp.exp(u)
    e10 = jnp.exp(10.0 * u)
    e1_ref[...] = e1
    n1_ref[...] = xs * e1
    e10_ref[...] = e10
    n10_ref[...] = xs * e10


def _expw(xs, bndr):
    e = xs.shape[0]
    be = 3200
    sp = pl.BlockSpec((be, F), lambda i: (i, 0))
    return pl.pallas_call(
        _expw_body,
        grid=(e // be,),
        in_specs=[sp, sp],
        out_specs=[sp, sp, sp, sp],
        out_shape=[jax.ShapeDtypeStruct((e, F), jnp.float32)] * 4,
    )(xs, bndr)


def _final_body(xm_ref, xv_ref, d1_ref, n1_ref, d10_ref, n10_ref,
                w0_ref, b0_ref, w1_ref, b1_ref, w2_ref, b2_ref, out_ref):
    xm = xm_ref[...]
    xv = xv_ref[...]
    sm1 = (n1_ref[0] + n1_ref[1]) / jnp.clip(d1_ref[0] + d1_ref[1], MIN_NORM, None)
    sm10 = (n10_ref[0] + n10_ref[1]) / jnp.clip(d10_ref[0] + d10_ref[1], MIN_NORM, None)
    cat = jnp.concatenate([xm, xv, sm1, sm10], axis=-1)
    h = jax.nn.gelu(_mm(cat, w0_ref[...]) + b0_ref[...])
    h = jax.nn.gelu(_mm(h, w1_ref[...]) + b1_ref[...])
    xa = _mm(h, w2_ref[...]) + b2_ref[...]
    xa = _proj(_expmap0(xa))
    xt = jax.nn.relu(_logmap0(xa))
    out_ref[...] = _proj(_expmap0(xt))


def _final(xm, xv, d1p, n1p, d10p, n10p, w0, b0, w1, b1, w2, b2):
    n = xm.shape[0]
    bn = 1000
    h = w0.shape[0]
    nb = pl.BlockSpec((bn, F), lambda i: (i, 0))
    pb = pl.BlockSpec((2, bn, F), lambda i: (0, i, 0))
    full = lambda shp: pl.BlockSpec(shp, lambda i: (0, 0))
    return pl.pallas_call(
        _final_body,
        grid=(n // bn,),
        in_specs=[nb, nb, pb, pb, pb, pb,
                  full((h, 4 * F)), full((1, h)),
                  full((h, h)), full((1, h)),
                  full((F, h)), full((1, F))],
        out_specs=nb,
        out_shape=jax.ShapeDtypeStruct((n, F), jnp.float32),
    )(xm, xv, d1p, n1p, d10p, n10p, w0, b0, w1, b1, w2, b2)


_CHUNK = 80
_GCHUNK = 128
_KB = 4
_SKB = 2


def _sc_mesh():
    return plsc.VectorSubcoreMesh(core_axis_name="c", subcore_axis_name="s",
                                  num_cores=NC, num_subcores=NS)


def _sc_gather(table, idx):
    e = idx.shape[0]
    per_w = e // NW
    n_full = per_w // _GCHUNK
    tail = per_w - n_full * _GCHUNK
    groups = n_full // _KB
    rem = n_full - groups * _KB
    d = table.shape[1]

    @functools.partial(
        pl.kernel,
        mesh=_sc_mesh(),
        out_type=jax.ShapeDtypeStruct((e, d), jnp.float32),
        scratch_types=[
            pltpu.VMEM((per_w,), jnp.int32),
            pltpu.VMEM((_KB, _GCHUNK, d), jnp.float32),
            pltpu.SemaphoreType.DMA,
            pltpu.SemaphoreType.DMA,
        ],
    )
    def k(table_hbm, idx_hbm, out_hbm, idx_v, rows_v, gsem, ssem):
        c = lax.axis_index("c")
        s = lax.axis_index("s")
        wid = s * NC + c
        base = wid * per_w
        pltpu.sync_copy(idx_hbm.at[wid], idx_v)

        def do_chunk_sync(off, size, b):
            pltpu.async_copy(table_hbm.at[idx_v.at[pl.ds(off, size)]],
                             rows_v.at[b, pl.ds(0, size)], gsem).wait()
            pltpu.sync_copy(rows_v.at[b, pl.ds(0, size)],
                            out_hbm.at[pl.ds(base + off, size)])

        def group(g, carry):
            j0 = g * _KB
            gds = [
                pltpu.async_copy(
                    table_hbm.at[idx_v.at[pl.ds((j0 + b) * _GCHUNK, _GCHUNK)]],
                    rows_v.at[b], gsem)
                for b in range(_KB)
            ]
            for dsc in gds:
                dsc.wait()
            sds = [
                pltpu.async_copy(
                    rows_v.at[b],
                    out_hbm.at[pl.ds(base + (j0 + b) * _GCHUNK, _GCHUNK)],
                    ssem)
                for b in range(_KB)
            ]
            for dsc in sds:
                dsc.wait()
            return carry

        lax.fori_loop(0, groups, group, 0)
        for t in range(rem):
            do_chunk_sync((groups * _KB + t) * _GCHUNK, _GCHUNK, t)
        if tail:
            do_chunk_sync(n_full * _GCHUNK, tail, 0)

    return k(table, idx.reshape(NW, per_w))


def _sc_scatter_add(vals, idx, zeros, n):
    e, d = vals.shape
    per_w = e // NW
    n_chunks = per_w // _CHUNK
    groups = n_chunks // _SKB
    rem = n_chunks - groups * _SKB
    rows_t = (n // NS) // 8 * 8
    tail = n - NS * rows_t

    @functools.partial(
        pl.kernel,
        mesh=_sc_mesh(),
        out_type=jax.ShapeDtypeStruct((NC, n, d), jnp.float32),
        scratch_types=[
            pltpu.VMEM((n_chunks, _CHUNK), jnp.int32),
            pltpu.VMEM((_SKB, _CHUNK, d), jnp.float32),
            pltpu.VMEM_SHARED((n, d), jnp.float32),
            pltpu.SemaphoreType.DMA,
            pltpu.SemaphoreType.DMA,
        ],
    )
    def k(vals_hbm, idx_hbm, zeros_hbm, out_hbm, idx_v, vals_v, acc_sh,
          lsem, scsem):
        c = lax.axis_index("c")
        s = lax.axis_index("s")
        wid = s * NC + c
        base = wid * per_w
        pltpu.sync_copy(zeros_hbm.at[pl.ds(s * rows_t, rows_t)],
                        acc_sh.at[pl.ds(s * rows_t, rows_t)])
        if tail:
            @pl.when(s == 0)
            def _():
                pltpu.sync_copy(zeros_hbm.at[pl.ds(NS * rows_t, tail)],
                                acc_sh.at[pl.ds(NS * rows_t, tail)])
        pltpu.sync_copy(idx_hbm.at[wid], idx_v)
        plsc.subcore_barrier()

        def group(g, carry):
            j0 = g * _SKB
            lds = [
                pltpu.async_copy(
                    vals_hbm.at[pl.ds(base + (j0 + b) * _CHUNK, _CHUNK)],
                    vals_v.at[b], lsem)
                for b in range(_SKB)
            ]
            for dsc in lds:
                dsc.wait()
            sds = [
                pltpu.async_copy(vals_v.at[b], acc_sh.at[idx_v.at[j0 + b]],
                                 scsem, add=True)
                for b in range(_SKB)
            ]
            for dsc in sds:
                dsc.wait()
            return carry

        lax.fori_loop(0, groups, group, 0)
        for t in range(rem):
            j = groups * _SKB + t
            pltpu.sync_copy(vals_hbm.at[pl.ds(base + j * _CHUNK, _CHUNK)],
                            vals_v.at[t])
            pltpu.sync_copy(vals_v.at[t], acc_sh.at[idx_v.at[j]], add=True)
        plsc.subcore_barrier()
        pltpu.sync_copy(acc_sh.at[pl.ds(s * rows_t, rows_t)],
                        out_hbm.at[c, pl.ds(s * rows_t, rows_t)])
        if tail:
            @pl.when(s == 0)
            def _():
                pltpu.sync_copy(acc_sh.at[pl.ds(NS * rows_t, tail)],
                                out_hbm.at[c, pl.ds(NS * rows_t, tail)])

    return k(vals, idx.reshape(NW, n_chunks, _CHUNK), zeros)


def kernel(x, adj, key, W, b, ec_W0, ec_b0, ec_W1, ec_b1, ec_W2, ec_b2,
           ag_W0, ag_b0, ag_W1, ag_b1, ag_W2, ag_b2):
    n = x.shape[0]
    s, r = adj[0], adj[1]

    bias = b.reshape(1, -1)
    bn = jnp.clip(jnp.sqrt(jnp.sum(bias * bias, -1, keepdims=True)), MIN_NORM, None)
    eb = jnp.tanh(bn) * bias / bn
    ebn = jnp.clip(jnp.sqrt(jnp.sum(eb * eb, -1, keepdims=True)), MIN_NORM, None)
    maxnorm = 1.0 - 4e-3
    hyp_bias = jnp.where(ebn > maxnorm, eb / ebn * maxnorm, eb)
    a_r = ec_W0[:, :F] - ec_W0[:, 2 * F:]
    a_s = ec_W0[:, F:2 * F] + ec_W0[:, 2 * F:]
    row = lambda v: v.reshape(1, -1)

    e_num = adj.shape[1]
    ht = _node_prep(x, W, hyp_bias)
    gcat = _sc_gather(ht, jnp.concatenate([s, r]))
    ghts, ghtr = gcat[:e_num], gcat[e_num:]
    xs, xsq = _edge_mlp(ghtr, ghts, a_r, a_s, row(ec_b0), ec_W1, row(ec_b1),
                        ec_W2, row(ec_b2))

    z128 = jnp.zeros((n, F), jnp.float32)
    z16 = jnp.zeros((n, 16), jnp.float32)
    ones16 = jnp.ones((adj.shape[1], 16), jnp.float32)
    sp = _sc_scatter_add(xs, r, z128, n)
    qp = _sc_scatter_add(xsq, r, z128, n)
    cp = _sc_scatter_add(ones16, r, z16, n)
    xm, xv, bnd = _stats(sp, qp, cp)

    bndr = _sc_gather(bnd, r)
    e1, n1, e10, n10 = _expw(xs, bndr)
    d1p = _sc_scatter_add(e1, r, z128, n)
    n1p = _sc_scatter_add(n1, r, z128, n)
    d10p = _sc_scatter_add(e10, r, z128, n)
    n10p = _sc_scatter_add(n10, r, z128, n)

    out = _final(xm, xv, d1p, n1p, d10p, n10p,
                 ag_W0, row(ag_b0), ag_W1, row(ag_b1), ag_W2, row(ag_b2))
    return (out, adj)


# speedup vs baseline: 3.8242x; 1.0320x over previous
import functools

import jax
import jax.numpy as jnp
from jax import lax
from jax.experimental import pallas as pl
from jax.experimental.pallas import tpu as pltpu
from jax.experimental.pallas import tpu_sc as plsc

F = 128
MIN_NORM = 1e-15

NC = 2
NS = 16
NW = NC * NS


def _rnorm(x):
    return jnp.clip(jnp.sqrt(jnp.sum(x * x, axis=-1, keepdims=True)), MIN_NORM, None)


def _artanh(z):
    z = jnp.clip(z, -1.0 + 1e-7, 1.0 - 1e-7)
    return 0.5 * jnp.log((1.0 + z) / (1.0 - z))


def _proj(x):
    maxnorm = 1.0 - 4e-3
    n = _rnorm(x)
    return jnp.where(n > maxnorm, x / n * maxnorm, x)


def _expmap0(u):
    un = _rnorm(u)
    return jnp.tanh(un) * u / un


def _logmap0(p):
    pn = _rnorm(p)
    return _artanh(pn) * p / pn


def _mm(a, b_t):
    return lax.dot_general(a, b_t, (((1,), (1,)), ((), ())),
                           preferred_element_type=jnp.float32)


def _node_prep_body(x_ref, w_ref, hb_ref, ht_ref):
    x = x_ref[...]
    w = w_ref[...]
    hb = hb_ref[...]
    xn = _rnorm(x)
    mx = _mm(x, w)
    mxn = _rnorm(mx)
    res = jnp.tanh(mxn / xn * _artanh(xn)) * mx / mxn
    res = _proj(res)
    x2 = jnp.sum(res * res, -1, keepdims=True)
    y2 = jnp.sum(hb * hb, -1, keepdims=True)
    xy = jnp.sum(res * hb, -1, keepdims=True)
    num = (1.0 + 2.0 * xy + y2) * res + (1.0 - x2) * hb
    den = 1.0 + 2.0 * xy + x2 * y2
    h = _proj(num / jnp.clip(den, MIN_NORM, None))
    ht_ref[...] = _logmap0(h)


def _node_prep(x, w, hyp_bias):
    n = x.shape[0]
    bn = 1000
    return pl.pallas_call(
        _node_prep_body,
        grid=(n // bn,),
        in_specs=[
            pl.BlockSpec((bn, F), lambda i: (i, 0)),
            pl.BlockSpec((F, F), lambda i: (0, 0)),
            pl.BlockSpec((1, F), lambda i: (0, 0)),
        ],
        out_specs=pl.BlockSpec((bn, F), lambda i: (i, 0)),
        out_shape=jax.ShapeDtypeStruct((n, F), jnp.float32),
    )(x, w, hyp_bias)


def _edge_mlp_body(gr_ref, gs_ref, ar_ref, as_ref, b0_ref, w1_ref, b1_ref,
                   w2_ref, b2_ref, xs_ref, xsq_ref):
    gr = gr_ref[...]
    gs = gs_ref[...]
    h1 = jax.nn.gelu(_mm(gr, ar_ref[...]) + _mm(gs, as_ref[...]) + b0_ref[...])
    h2 = jax.nn.gelu(_mm(h1, w1_ref[...]) + b1_ref[...])
    xs = _mm(h2, w2_ref[...]) + b2_ref[...]
    xs_ref[...] = xs
    xsq_ref[...] = xs * xs


def _edge_mlp(gr, gs, a_r, a_s, b0, w1, b1, w2, b2):
    e = gr.shape[0]
    be = 3200
    h = a_r.shape[0]
    full = lambda shp: pl.BlockSpec(shp, lambda i: (0, 0))
    return pl.pallas_call(
        _edge_mlp_body,
        grid=(e // be,),
        in_specs=[
            pl.BlockSpec((be, F), lambda i: (i, 0)),
            pl.BlockSpec((be, F), lambda i: (i, 0)),
            full((h, F)), full((h, F)), full((1, h)),
            full((h, h)), full((1, h)),
            full((F, h)), full((1, F)),
        ],
        out_specs=[
            pl.BlockSpec((be, F), lambda i: (i, 0)),
            pl.BlockSpec((be, F), lambda i: (i, 0)),
        ],
        out_shape=[
            jax.ShapeDtypeStruct((e, F), jnp.float32),
            jax.ShapeDtypeStruct((e, F), jnp.float32),
        ],
    )(gr, gs, a_r, a_s, b0, w1, b1, w2, b2)


def _stats_body(sp_ref, qp_ref, cp_ref, xm_ref, xv_ref, bnd_ref):
    s = sp_ref[0] + sp_ref[1]
    q = qp_ref[0] + qp_ref[1]
    cnt = (cp_ref[0] + cp_ref[1])[:, :1]
    cc = jnp.clip(cnt, 1.0, None)
    xm = s / cc
    xv = jnp.maximum(q / cc - xm * xm, 0.0)
    xm_ref[...] = xm
    xv_ref[...] = xv
    bnd_ref[...] = xm + jnp.sqrt(xv * jnp.clip(cnt - 1.0, 0.0, None))


def _stats(sp, qp, cp):
    n = sp.shape[1]
    bn = 1000
    o = pl.BlockSpec((bn, F), lambda i: (i, 0))
    return pl.pallas_call(
        _stats_body,
        grid=(n // bn,),
        in_specs=[
            pl.BlockSpec((2, bn, F), lambda i: (0, i, 0)),
            pl.BlockSpec((2, bn, F), lambda i: (0, i, 0)),
            pl.BlockSpec((2, bn, 16), lambda i: (0, i, 0)),
        ],
        out_specs=[o, o, o],
        out_shape=[jax.ShapeDtypeStruct((n, F), jnp.float32)] * 3,
    )(sp, qp, cp)


def _expw_body(xs_ref, bd_ref, e1_ref, n1_ref, e10_ref, n10_ref):
    xs = xs_ref[...]
    u = xs - bd_ref[...]
    e1 = jnp.exp(u)
    e10 = jnp.exp(10.0 * u)
    e1_ref[...] = e1
    n1_ref[...] = xs * e1
    e10_ref[...] = e10
    n10_ref[...] = xs * e10


def _expw(xs, bndr):
    e = xs.shape[0]
    be = 3200
    sp = pl.BlockSpec((be, F), lambda i: (i, 0))
    return pl.pallas_call(
        _expw_body,
        grid=(e // be,),
        in_specs=[sp, sp],
        out_specs=[sp, sp, sp, sp],
        out_shape=[jax.ShapeDtypeStruct((e, F), jnp.float32)] * 4,
    )(xs, bndr)


def _final_body(xm_ref, xv_ref, d1_ref, n1_ref, d10_ref, n10_ref,
                w0_ref, b0_ref, w1_ref, b1_ref, w2_ref, b2_ref, out_ref):
    xm = xm_ref[...]
    xv = xv_ref[...]
    sm1 = (n1_ref[0] + n1_ref[1]) / jnp.clip(d1_ref[0] + d1_ref[1], MIN_NORM, None)
    sm10 = (n10_ref[0] + n10_ref[1]) / jnp.clip(d10_ref[0] + d10_ref[1], MIN_NORM, None)
    cat = jnp.concatenate([xm, xv, sm1, sm10], axis=-1)
    h = jax.nn.gelu(_mm(cat, w0_ref[...]) + b0_ref[...])
    h = jax.nn.gelu(_mm(h, w1_ref[...]) + b1_ref[...])
    xa = _mm(h, w2_ref[...]) + b2_ref[...]
    xa = _proj(_expmap0(xa))
    xt = jax.nn.relu(_logmap0(xa))
    out_ref[...] = _proj(_expmap0(xt))


def _final(xm, xv, d1p, n1p, d10p, n10p, w0, b0, w1, b1, w2, b2):
    n = xm.shape[0]
    bn = 1000
    h = w0.shape[0]
    nb = pl.BlockSpec((bn, F), lambda i: (i, 0))
    pb = pl.BlockSpec((2, bn, F), lambda i: (0, i, 0))
    full = lambda shp: pl.BlockSpec(shp, lambda i: (0, 0))
    return pl.pallas_call(
        _final_body,
        grid=(n // bn,),
        in_specs=[nb, nb, pb, pb, pb, pb,
                  full((h, 4 * F)), full((1, h)),
                  full((h, h)), full((1, h)),
                  full((F, h)), full((1, F))],
        out_specs=nb,
        out_shape=jax.ShapeDtypeStruct((n, F), jnp.float32),
    )(xm, xv, d1p, n1p, d10p, n10p, w0, b0, w1, b1, w2, b2)


_CHUNK = 80
_GCHUNK = 128
_KB = 5
_SKB = 3


def _sc_mesh():
    return plsc.VectorSubcoreMesh(core_axis_name="c", subcore_axis_name="s",
                                  num_cores=NC, num_subcores=NS)


def _sc_gather(table, idx):
    e = idx.shape[0]
    per_w = e // NW
    n_full = per_w // _GCHUNK
    tail = per_w - n_full * _GCHUNK
    groups = n_full // _KB
    rem = n_full - groups * _KB
    d = table.shape[1]

    @functools.partial(
        pl.kernel,
        mesh=_sc_mesh(),
        out_type=jax.ShapeDtypeStruct((e, d), jnp.float32),
        scratch_types=[
            pltpu.VMEM((per_w,), jnp.int32),
            pltpu.VMEM((_KB, _GCHUNK, d), jnp.float32),
            pltpu.SemaphoreType.DMA,
            pltpu.SemaphoreType.DMA,
        ],
    )
    def k(table_hbm, idx_hbm, out_hbm, idx_v, rows_v, gsem, ssem):
        c = lax.axis_index("c")
        s = lax.axis_index("s")
        wid = s * NC + c
        base = wid * per_w
        pltpu.sync_copy(idx_hbm.at[wid], idx_v)

        def do_chunk_sync(off, size, b):
            pltpu.async_copy(table_hbm.at[idx_v.at[pl.ds(off, size)]],
                             rows_v.at[b, pl.ds(0, size)], gsem).wait()
            pltpu.sync_copy(rows_v.at[b, pl.ds(0, size)],
                            out_hbm.at[pl.ds(base + off, size)])

        def group(g, carry):
            j0 = g * _KB
            gds = [
                pltpu.async_copy(
                    table_hbm.at[idx_v.at[pl.ds((j0 + b) * _GCHUNK, _GCHUNK)]],
                    rows_v.at[b], gsem)
                for b in range(_KB)
            ]
            for dsc in gds:
                dsc.wait()
            sds = [
                pltpu.async_copy(
                    rows_v.at[b],
                    out_hbm.at[pl.ds(base + (j0 + b) * _GCHUNK, _GCHUNK)],
                    ssem)
                for b in range(_KB)
            ]
            for dsc in sds:
                dsc.wait()
            return carry

        lax.fori_loop(0, groups, group, 0)
        for t in range(rem):
            do_chunk_sync((groups * _KB + t) * _GCHUNK, _GCHUNK, t)
        if tail:
            do_chunk_sync(n_full * _GCHUNK, tail, 0)

    return k(table, idx.reshape(NW, per_w))


def _sc_scatter_add(vals, idx, zeros, n):
    e, d = vals.shape
    per_w = e // NW
    n_chunks = per_w // _CHUNK
    groups = n_chunks // _SKB
    rem = n_chunks - groups * _SKB
    rows_t = (n // NS) // 8 * 8
    tail = n - NS * rows_t

    @functools.partial(
        pl.kernel,
        mesh=_sc_mesh(),
        out_type=jax.ShapeDtypeStruct((NC, n, d), jnp.float32),
        scratch_types=[
            pltpu.VMEM((n_chunks, _CHUNK), jnp.int32),
            pltpu.VMEM((_SKB, _CHUNK, d), jnp.float32),
            pltpu.VMEM_SHARED((n, d), jnp.float32),
            pltpu.SemaphoreType.DMA,
            pltpu.SemaphoreType.DMA,
        ],
    )
    def k(vals_hbm, idx_hbm, zeros_hbm, out_hbm, idx_v, vals_v, acc_sh,
          lsem, scsem):
        c = lax.axis_index("c")
        s = lax.axis_index("s")
        wid = s * NC + c
        base = wid * per_w
        pltpu.sync_copy(zeros_hbm.at[pl.ds(s * rows_t, rows_t)],
                        acc_sh.at[pl.ds(s * rows_t, rows_t)])
        if tail:
            @pl.when(s == 0)
            def _():
                pltpu.sync_copy(zeros_hbm.at[pl.ds(NS * rows_t, tail)],
                                acc_sh.at[pl.ds(NS * rows_t, tail)])
        pltpu.sync_copy(idx_hbm.at[wid], idx_v)
        plsc.subcore_barrier()

        def group(g, carry):
            j0 = g * _SKB
            lds = [
                pltpu.async_copy(
                    vals_hbm.at[pl.ds(base + (j0 + b) * _CHUNK, _CHUNK)],
                    vals_v.at[b], lsem)
                for b in range(_SKB)
            ]
            for dsc in lds:
                dsc.wait()
            sds = [
                pltpu.async_copy(vals_v.at[b], acc_sh.at[idx_v.at[j0 + b]],
                                 scsem, add=True)
                for b in range(_SKB)
            ]
            for dsc in sds:
                dsc.wait()
            return carry

        lax.fori_loop(0, groups, group, 0)
        for t in range(rem):
            j = groups * _SKB + t
            pltpu.sync_copy(vals_hbm.at[pl.ds(base + j * _CHUNK, _CHUNK)],
                            vals_v.at[t])
            pltpu.sync_copy(vals_v.at[t], acc_sh.at[idx_v.at[j]], add=True)
        plsc.subcore_barrier()
        pltpu.sync_copy(acc_sh.at[pl.ds(s * rows_t, rows_t)],
                        out_hbm.at[c, pl.ds(s * rows_t, rows_t)])
        if tail:
            @pl.when(s == 0)
            def _():
                pltpu.sync_copy(acc_sh.at[pl.ds(NS * rows_t, tail)],
                                out_hbm.at[c, pl.ds(NS * rows_t, tail)])

    return k(vals, idx.reshape(NW, n_chunks, _CHUNK), zeros)


def kernel(x, adj, key, W, b, ec_W0, ec_b0, ec_W1, ec_b1, ec_W2, ec_b2,
           ag_W0, ag_b0, ag_W1, ag_b1, ag_W2, ag_b2):
    n = x.shape[0]
    s, r = adj[0], adj[1]

    bias = b.reshape(1, -1)
    bn = jnp.clip(jnp.sqrt(jnp.sum(bias * bias, -1, keepdims=True)), MIN_NORM, None)
    eb = jnp.tanh(bn) * bias / bn
    ebn = jnp.clip(jnp.sqrt(jnp.sum(eb * eb, -1, keepdims=True)), MIN_NORM, None)
    maxnorm = 1.0 - 4e-3
    hyp_bias = jnp.where(ebn > maxnorm, eb / ebn * maxnorm, eb)
    a_r = ec_W0[:, :F] - ec_W0[:, 2 * F:]
    a_s = ec_W0[:, F:2 * F] + ec_W0[:, 2 * F:]
    row = lambda v: v.reshape(1, -1)

    e_num = adj.shape[1]
    n16 = jnp.zeros((n, 16), jnp.float32)
    ones16 = jnp.ones((e_num, 16), jnp.float32)
    cp = _sc_scatter_add(ones16, r, n16, n)
    ht = _node_prep(x, W, hyp_bias)
    gcat = _sc_gather(ht, jnp.concatenate([s, r]))
    ghts, ghtr = gcat[:e_num], gcat[e_num:]
    xs, xsq = _edge_mlp(ghtr, ghts, a_r, a_s, row(ec_b0), ec_W1, row(ec_b1),
                        ec_W2, row(ec_b2))

    z128 = jnp.zeros((n, F), jnp.float32)
    sp = _sc_scatter_add(xs, r, z128, n)
    qp = _sc_scatter_add(xsq, r, z128, n)
    xm, xv, bnd = _stats(sp, qp, cp)

    bndr = _sc_gather(bnd, r)
    e1, n1, e10, n10 = _expw(xs, bndr)
    d1p = _sc_scatter_add(e1, r, z128, n)
    n1p = _sc_scatter_add(n1, r, z128, n)
    d10p = _sc_scatter_add(e10, r, z128, n)
    n10p = _sc_scatter_add(n10, r, z128, n)

    out = _final(xm, xv, d1p, n1p, d10p, n10p,
                 ag_W0, row(ag_b0), ag_W1, row(ag_b1), ag_W2, row(ag_b2))
    return (out, adj)


# speedup vs baseline: 3.9868x; 1.0425x over previous
import functools

import jax
import jax.numpy as jnp
from jax import lax
from jax.experimental import pallas as pl
from jax.experimental.pallas import tpu as pltpu
from jax.experimental.pallas import tpu_sc as plsc

F = 128
MIN_NORM = 1e-15

NC = 2
NS = 16
NW = NC * NS


def _rnorm(x):
    return jnp.clip(jnp.sqrt(jnp.sum(x * x, axis=-1, keepdims=True)), MIN_NORM, None)


def _artanh(z):
    z = jnp.clip(z, -1.0 + 1e-7, 1.0 - 1e-7)
    return 0.5 * jnp.log((1.0 + z) / (1.0 - z))


def _proj(x):
    maxnorm = 1.0 - 4e-3
    n = _rnorm(x)
    return jnp.where(n > maxnorm, x / n * maxnorm, x)


def _expmap0(u):
    un = _rnorm(u)
    return jnp.tanh(un) * u / un


def _logmap0(p):
    pn = _rnorm(p)
    return _artanh(pn) * p / pn


def _mm(a, b_t):
    return lax.dot_general(a, b_t, (((1,), (1,)), ((), ())),
                           preferred_element_type=jnp.float32)


def _node_prep_body(x_ref, w_ref, hb_ref, ht_ref):
    x = x_ref[...]
    w = w_ref[...]
    hb = hb_ref[...]
    xn = _rnorm(x)
    mx = _mm(x, w)
    mxn = _rnorm(mx)
    res = jnp.tanh(mxn / xn * _artanh(xn)) * mx / mxn
    res = _proj(res)
    x2 = jnp.sum(res * res, -1, keepdims=True)
    y2 = jnp.sum(hb * hb, -1, keepdims=True)
    xy = jnp.sum(res * hb, -1, keepdims=True)
    num = (1.0 + 2.0 * xy + y2) * res + (1.0 - x2) * hb
    den = 1.0 + 2.0 * xy + x2 * y2
    h = _proj(num / jnp.clip(den, MIN_NORM, None))
    ht_ref[...] = _logmap0(h)


def _node_prep(x, w, hyp_bias):
    n = x.shape[0]
    bn = 1000
    return pl.pallas_call(
        _node_prep_body,
        grid=(n // bn,),
        in_specs=[
            pl.BlockSpec((bn, F), lambda i: (i, 0)),
            pl.BlockSpec((F, F), lambda i: (0, 0)),
            pl.BlockSpec((1, F), lambda i: (0, 0)),
        ],
        out_specs=pl.BlockSpec((bn, F), lambda i: (i, 0)),
        out_shape=jax.ShapeDtypeStruct((n, F), jnp.float32),
    )(x, w, hyp_bias)


def _edge_mlp_body(gr_ref, gs_ref, ar_ref, as_ref, b0_ref, w1_ref, b1_ref,
                   w2_ref, b2_ref, xs1_ref, xsq_ref):
    gr = gr_ref[...]
    gs = gs_ref[...]
    h1 = jax.nn.gelu(_mm(gr, ar_ref[...]) + _mm(gs, as_ref[...]) + b0_ref[...])
    h2 = jax.nn.gelu(_mm(h1, w1_ref[...]) + b1_ref[...])
    xs = _mm(h2, w2_ref[...]) + b2_ref[...]
    xs1_ref[...] = xs
    xsq_ref[...] = xs * xs


def _edge_mlp(gr, gs, a_r, a_s, b0, w1, b1, w2, b2):
    e = gr.shape[0]
    be = 3200
    h = a_r.shape[0]
    full = lambda shp: pl.BlockSpec(shp, lambda i: (0, 0))
    return pl.pallas_call(
        _edge_mlp_body,
        grid=(e // be,),
        in_specs=[
            pl.BlockSpec((be, F), lambda i: (i, 0)),
            pl.BlockSpec((be, F), lambda i: (i, 0)),
            full((h, F)), full((h, F)), full((1, h)),
            full((h, h)), full((1, h)),
            full((F, h)), full((1, F)),
        ],
        out_specs=[
            pl.BlockSpec((be, F), lambda i: (i, 0)),
            pl.BlockSpec((be, F), lambda i: (i, 0)),
        ],
        out_shape=[
            jax.ShapeDtypeStruct((e, F), jnp.float32),
            jax.ShapeDtypeStruct((e, F), jnp.float32),
        ],
    )(gr, gs, a_r, a_s, b0, w1, b1, w2, b2)


def _stats_body(sp_ref, qp_ref, cp_ref, xm_ref, xv_ref, bnd_ref):
    s = sp_ref[0] + sp_ref[1]
    q = qp_ref[0] + qp_ref[1]
    cnt = (cp_ref[0] + cp_ref[1])[:, :1]
    cc = jnp.clip(cnt, 1.0, None)
    xm = s / cc
    xv = jnp.maximum(q / cc - xm * xm, 0.0)
    xm_ref[...] = xm
    xv_ref[...] = xv
    bnd_ref[...] = xm + jnp.sqrt(xv * jnp.clip(cnt - 1.0, 0.0, None))


def _stats(sp, qp, cp):
    n = sp.shape[1]
    bn = 1000
    o = pl.BlockSpec((bn, F), lambda i: (i, 0))
    return pl.pallas_call(
        _stats_body,
        grid=(n // bn,),
        in_specs=[
            pl.BlockSpec((2, bn, F), lambda i: (0, i, 0)),
            pl.BlockSpec((2, bn, F), lambda i: (0, i, 0)),
            pl.BlockSpec((2, bn, 16), lambda i: (0, i, 0)),
        ],
        out_specs=[o, o, o],
        out_shape=[jax.ShapeDtypeStruct((n, F), jnp.float32)] * 3,
    )(sp, qp, cp)


def _expw_body(xs_ref, bd_ref, e1_ref, n1_ref, e10_ref, n10_ref):
    xs = xs_ref[...]
    u = xs - bd_ref[...]
    e1 = jnp.exp(u)
    e10 = jnp.exp(10.0 * u)
    e1_ref[...] = e1
    n1_ref[...] = xs * e1
    e10_ref[...] = e10
    n10_ref[...] = xs * e10


def _expw(xs, bndr):
    e = xs.shape[0]
    be = 3200
    sp = pl.BlockSpec((be, F), lambda i: (i, 0))
    return pl.pallas_call(
        _expw_body,
        grid=(e // be,),
        in_specs=[sp, sp],
        out_specs=[sp, sp, sp, sp],
        out_shape=[jax.ShapeDtypeStruct((e, F), jnp.float32)] * 4,
    )(xs, bndr)


def _final_body(xm_ref, xv_ref, d1_ref, n1_ref, d10_ref, n10_ref,
                w0_ref, b0_ref, w1_ref, b1_ref, w2_ref, b2_ref, out_ref):
    xm = xm_ref[...]
    xv = xv_ref[...]
    sm1 = (n1_ref[0] + n1_ref[1]) / jnp.clip(d1_ref[0] + d1_ref[1], MIN_NORM, None)
    sm10 = (n10_ref[0] + n10_ref[1]) / jnp.clip(d10_ref[0] + d10_ref[1], MIN_NORM, None)
    cat = jnp.concatenate([xm, xv, sm1, sm10], axis=-1)
    h = jax.nn.gelu(_mm(cat, w0_ref[...]) + b0_ref[...])
    h = jax.nn.gelu(_mm(h, w1_ref[...]) + b1_ref[...])
    xa = _mm(h, w2_ref[...]) + b2_ref[...]
    xa = _proj(_expmap0(xa))
    xt = jax.nn.relu(_logmap0(xa))
    out_ref[...] = _proj(_expmap0(xt))


def _final(xm, xv, d1p, n1p, d10p, n10p, w0, b0, w1, b1, w2, b2):
    n = xm.shape[0]
    bn = 1000
    h = w0.shape[0]
    nb = pl.BlockSpec((bn, F), lambda i: (i, 0))
    pb = pl.BlockSpec((2, bn, F), lambda i: (0, i, 0))
    full = lambda shp: pl.BlockSpec(shp, lambda i: (0, 0))
    return pl.pallas_call(
        _final_body,
        grid=(n // bn,),
        in_specs=[nb, nb, pb, pb, pb, pb,
                  full((h, 4 * F)), full((1, h)),
                  full((h, h)), full((1, h)),
                  full((F, h)), full((1, F))],
        out_specs=nb,
        out_shape=jax.ShapeDtypeStruct((n, F), jnp.float32),
    )(xm, xv, d1p, n1p, d10p, n10p, w0, b0, w1, b1, w2, b2)


_CHUNK = 80
_GCHUNK = 128
_KB = 5
_SKB = 3


def _sc_mesh():
    return plsc.VectorSubcoreMesh(core_axis_name="c", subcore_axis_name="s",
                                  num_cores=NC, num_subcores=NS)


def _sc_gather(table, idx):
    e = idx.shape[0]
    per_w = e // NW
    n_full = per_w // _GCHUNK
    tail = per_w - n_full * _GCHUNK
    groups = n_full // _KB
    rem = n_full - groups * _KB
    d = table.shape[1]

    @functools.partial(
        pl.kernel,
        mesh=_sc_mesh(),
        out_type=jax.ShapeDtypeStruct((e, d), jnp.float32),
        scratch_types=[
            pltpu.VMEM((per_w,), jnp.int32),
            pltpu.VMEM((_KB, _GCHUNK, d), jnp.float32),
            pltpu.SemaphoreType.DMA,
            pltpu.SemaphoreType.DMA,
        ],
    )
    def k(table_hbm, idx_hbm, out_hbm, idx_v, rows_v, gsem, ssem):
        c = lax.axis_index("c")
        s = lax.axis_index("s")
        wid = s * NC + c
        base = wid * per_w
        pltpu.sync_copy(idx_hbm.at[wid], idx_v)

        def do_chunk_sync(off, size, b):
            pltpu.async_copy(table_hbm.at[idx_v.at[pl.ds(off, size)]],
                             rows_v.at[b, pl.ds(0, size)], gsem).wait()
            pltpu.sync_copy(rows_v.at[b, pl.ds(0, size)],
                            out_hbm.at[pl.ds(base + off, size)])

        def group(g, carry):
            j0 = g * _KB
            gds = [
                pltpu.async_copy(
                    table_hbm.at[idx_v.at[pl.ds((j0 + b) * _GCHUNK, _GCHUNK)]],
                    rows_v.at[b], gsem)
                for b in range(_KB)
            ]
            for dsc in gds:
                dsc.wait()
            sds = [
                pltpu.async_copy(
                    rows_v.at[b],
                    out_hbm.at[pl.ds(base + (j0 + b) * _GCHUNK, _GCHUNK)],
                    ssem)
                for b in range(_KB)
            ]
            for dsc in sds:
                dsc.wait()
            return carry

        lax.fori_loop(0, groups, group, 0)
        for t in range(rem):
            do_chunk_sync((groups * _KB + t) * _GCHUNK, _GCHUNK, t)
        if tail:
            do_chunk_sync(n_full * _GCHUNK, tail, 0)

    return k(table, idx.reshape(NW, per_w))


def _sc_scatter_add(vals, idx, zeros, n, skb=_SKB):
    e, d = vals.shape
    per_w = e // NW
    n_chunks = per_w // _CHUNK
    groups = n_chunks // skb
    rem = n_chunks - groups * skb
    rows_t = (n // NS) // 8 * 8
    tail = n - NS * rows_t

    @functools.partial(
        pl.kernel,
        mesh=_sc_mesh(),
        out_type=jax.ShapeDtypeStruct((NC, n, d), jnp.float32),
        scratch_types=[
            pltpu.VMEM((n_chunks, _CHUNK), jnp.int32),
            pltpu.VMEM((skb, _CHUNK, d), jnp.float32),
            pltpu.VMEM_SHARED((n, d), jnp.float32),
            pltpu.SemaphoreType.DMA,
            pltpu.SemaphoreType.DMA,
        ],
    )
    def k(vals_hbm, idx_hbm, zeros_hbm, out_hbm, idx_v, vals_v, acc_sh,
          lsem, scsem):
        c = lax.axis_index("c")
        s = lax.axis_index("s")
        wid = s * NC + c
        base = wid * per_w
        pltpu.sync_copy(zeros_hbm.at[pl.ds(s * rows_t, rows_t)],
                        acc_sh.at[pl.ds(s * rows_t, rows_t)])
        if tail:
            @pl.when(s == 0)
            def _():
                pltpu.sync_copy(zeros_hbm.at[pl.ds(NS * rows_t, tail)],
                                acc_sh.at[pl.ds(NS * rows_t, tail)])
        pltpu.sync_copy(idx_hbm.at[wid], idx_v)
        plsc.subcore_barrier()

        def group(g, carry):
            j0 = g * skb
            lds = [
                pltpu.async_copy(
                    vals_hbm.at[pl.ds(base + (j0 + b) * _CHUNK, _CHUNK)],
                    vals_v.at[b], lsem)
                for b in range(skb)
            ]
            for dsc in lds:
                dsc.wait()
            sds = [
                pltpu.async_copy(vals_v.at[b], acc_sh.at[idx_v.at[j0 + b]],
                                 scsem, add=True)
                for b in range(skb)
            ]
            for dsc in sds:
                dsc.wait()
            return carry

        lax.fori_loop(0, groups, group, 0)
        for t in range(rem):
            j = groups * skb + t
            pltpu.sync_copy(vals_hbm.at[pl.ds(base + j * _CHUNK, _CHUNK)],
                            vals_v.at[t])
            pltpu.sync_copy(vals_v.at[t], acc_sh.at[idx_v.at[j]], add=True)
        plsc.subcore_barrier()
        pltpu.sync_copy(acc_sh.at[pl.ds(s * rows_t, rows_t)],
                        out_hbm.at[c, pl.ds(s * rows_t, rows_t)])
        if tail:
            @pl.when(s == 0)
            def _():
                pltpu.sync_copy(acc_sh.at[pl.ds(NS * rows_t, tail)],
                                out_hbm.at[c, pl.ds(NS * rows_t, tail)])

    return k(vals, idx.reshape(NW, n_chunks, _CHUNK), zeros)


def _sc_count(idx, ones_chunk, zeros, n):
    nw_, n_chunks, chunk = idx.shape
    d = ones_chunk.shape[1]
    kb = 8
    groups = n_chunks // kb
    rem = n_chunks - groups * kb
    rows_t = (n // NS) // 8 * 8
    tail = n - NS * rows_t

    @functools.partial(
        pl.kernel,
        mesh=_sc_mesh(),
        out_type=jax.ShapeDtypeStruct((NC, n, d), jnp.float32),
        scratch_types=[
            pltpu.VMEM((n_chunks, chunk), jnp.int32),
            pltpu.VMEM((chunk, d), jnp.float32),
            pltpu.VMEM_SHARED((n, d), jnp.float32),
            pltpu.SemaphoreType.DMA,
        ],
    )
    def k(idx_hbm, ones_hbm, zeros_hbm, out_hbm, idx_v, ones_v, acc_sh, scsem):
        c = lax.axis_index("c")
        s = lax.axis_index("s")
        wid = s * NC + c
        pltpu.sync_copy(zeros_hbm.at[pl.ds(s * rows_t, rows_t)],
                        acc_sh.at[pl.ds(s * rows_t, rows_t)])
        if tail:
            @pl.when(s == 0)
            def _():
                pltpu.sync_copy(zeros_hbm.at[pl.ds(NS * rows_t, tail)],
                                acc_sh.at[pl.ds(NS * rows_t, tail)])
        pltpu.sync_copy(ones_hbm, ones_v)
        pltpu.sync_copy(idx_hbm.at[wid], idx_v)
        plsc.subcore_barrier()

        def group(g, carry):
            j0 = g * kb
            sds = [
                pltpu.async_copy(ones_v, acc_sh.at[idx_v.at[j0 + b]],
                                 scsem, add=True)
                for b in range(kb)
            ]
            for dsc in sds:
                dsc.wait()
            return carry

        lax.fori_loop(0, groups, group, 0)
        for t in range(rem):
            pltpu.sync_copy(ones_v, acc_sh.at[idx_v.at[groups * kb + t]],
                            add=True)
        plsc.subcore_barrier()
        pltpu.sync_copy(acc_sh.at[pl.ds(s * rows_t, rows_t)],
                        out_hbm.at[c, pl.ds(s * rows_t, rows_t)])
        if tail:
            @pl.when(s == 0)
            def _():
                pltpu.sync_copy(acc_sh.at[pl.ds(NS * rows_t, tail)],
                                out_hbm.at[c, pl.ds(NS * rows_t, tail)])

    return k(idx, ones_chunk, zeros)


def kernel(x, adj, key, W, b, ec_W0, ec_b0, ec_W1, ec_b1, ec_W2, ec_b2,
           ag_W0, ag_b0, ag_W1, ag_b1, ag_W2, ag_b2):
    n = x.shape[0]
    s, r = adj[0], adj[1]

    bias = b.reshape(1, -1)
    bn = jnp.clip(jnp.sqrt(jnp.sum(bias * bias, -1, keepdims=True)), MIN_NORM, None)
    eb = jnp.tanh(bn) * bias / bn
    ebn = jnp.clip(jnp.sqrt(jnp.sum(eb * eb, -1, keepdims=True)), MIN_NORM, None)
    maxnorm = 1.0 - 4e-3
    hyp_bias = jnp.where(ebn > maxnorm, eb / ebn * maxnorm, eb)
    a_r = ec_W0[:, :F] - ec_W0[:, 2 * F:]
    a_s = ec_W0[:, F:2 * F] + ec_W0[:, 2 * F:]
    row = lambda v: v.reshape(1, -1)

    e_num = adj.shape[1]
    ht = _node_prep(x, W, hyp_bias)
    gcat = _sc_gather(ht, jnp.concatenate([s, r]))
    ghts, ghtr = gcat[:e_num], gcat[e_num:]
    xs, xsq = _edge_mlp(ghtr, ghts, a_r, a_s, row(ec_b0), ec_W1, row(ec_b1),
                        ec_W2, row(ec_b2))

    z128 = jnp.zeros((n, F), jnp.float32)
    z16 = jnp.zeros((n, 16), jnp.float32)
    ones16 = jnp.ones((_CHUNK, 16), jnp.float32)
    cp = _sc_count(r.reshape(NW, -1, _CHUNK), ones16, z16, n)
    sp = _sc_scatter_add(xs, r, z128, n)
    qp = _sc_scatter_add(xsq, r, z128, n)
    xm, xv, bnd = _stats(sp, qp, cp)

    bndr = _sc_gather(bnd, r)
    e1, n1, e10, n10 = _expw(xs, bndr)
    d1p = _sc_scatter_add(e1, r, z128, n)
    n1p = _sc_scatter_add(n1, r, z128, n)
    d10p = _sc_scatter_add(e10, r, z128, n)
    n10p = _sc_scatter_add(n10, r, z128, n)

    out = _final(xm, xv, d1p, n1p, d10p, n10p,
                 ag_W0, row(ag_b0), ag_W1, row(ag_b1), ag_W2, row(ag_b2))
    return (out, adj)


# speedup vs baseline: 4.0371x; 1.0126x over previous
import functools

import jax
import jax.numpy as jnp
from jax import lax
from jax.experimental import pallas as pl
from jax.experimental.pallas import tpu as pltpu
from jax.experimental.pallas import tpu_sc as plsc

F = 128
MIN_NORM = 1e-15

NC = 2
NS = 16
NW = NC * NS


def _rnorm(x):
    return jnp.clip(jnp.sqrt(jnp.sum(x * x, axis=-1, keepdims=True)), MIN_NORM, None)


def _artanh(z):
    z = jnp.clip(z, -1.0 + 1e-7, 1.0 - 1e-7)
    return 0.5 * jnp.log((1.0 + z) / (1.0 - z))


def _proj(x):
    maxnorm = 1.0 - 4e-3
    n = _rnorm(x)
    return jnp.where(n > maxnorm, x / n * maxnorm, x)


def _expmap0(u):
    un = _rnorm(u)
    return jnp.tanh(un) * u / un


def _logmap0(p):
    pn = _rnorm(p)
    return _artanh(pn) * p / pn


def _mm(a, b_t):
    return lax.dot_general(a, b_t, (((1,), (1,)), ((), ())),
                           preferred_element_type=jnp.float32)


def _node_prep_body(x_ref, w_ref, hb_ref, ht_ref):
    x = x_ref[...]
    w = w_ref[...]
    hb = hb_ref[...]
    xn = _rnorm(x)
    mx = _mm(x, w)
    mxn = _rnorm(mx)
    res = jnp.tanh(mxn / xn * _artanh(xn)) * mx / mxn
    res = _proj(res)
    x2 = jnp.sum(res * res, -1, keepdims=True)
    y2 = jnp.sum(hb * hb, -1, keepdims=True)
    xy = jnp.sum(res * hb, -1, keepdims=True)
    num = (1.0 + 2.0 * xy + y2) * res + (1.0 - x2) * hb
    den = 1.0 + 2.0 * xy + x2 * y2
    h = _proj(num / jnp.clip(den, MIN_NORM, None))
    ht_ref[...] = _logmap0(h)


def _node_prep(x, w, hyp_bias):
    n = x.shape[0]
    bn = 1000
    return pl.pallas_call(
        _node_prep_body,
        grid=(n // bn,),
        in_specs=[
            pl.BlockSpec((bn, F), lambda i: (i, 0)),
            pl.BlockSpec((F, F), lambda i: (0, 0)),
            pl.BlockSpec((1, F), lambda i: (0, 0)),
        ],
        out_specs=pl.BlockSpec((bn, F), lambda i: (i, 0)),
        out_shape=jax.ShapeDtypeStruct((n, F), jnp.float32),
    )(x, w, hyp_bias)


def _edge_mlp_body(gr_ref, gs_ref, ar_ref, as_ref, b0_ref, w1_ref, b1_ref,
                   w2_ref, b2_ref, xs1_ref, xsq_ref):
    gr = gr_ref[...]
    gs = gs_ref[...]
    h1 = jax.nn.gelu(_mm(gr, ar_ref[...]) + _mm(gs, as_ref[...]) + b0_ref[...])
    h2 = jax.nn.gelu(_mm(h1, w1_ref[...]) + b1_ref[...])
    xs = _mm(h2, w2_ref[...]) + b2_ref[...]
    xs1_ref[...] = xs
    xsq_ref[...] = xs * xs


def _edge_mlp(gr, gs, a_r, a_s, b0, w1, b1, w2, b2):
    e = gr.shape[0]
    be = 3200
    h = a_r.shape[0]
    full = lambda shp: pl.BlockSpec(shp, lambda i: (0, 0))
    return pl.pallas_call(
        _edge_mlp_body,
        grid=(e // be,),
        in_specs=[
            pl.BlockSpec((be, F), lambda i: (i, 0)),
            pl.BlockSpec((be, F), lambda i: (i, 0)),
            full((h, F)), full((h, F)), full((1, h)),
            full((h, h)), full((1, h)),
            full((F, h)), full((1, F)),
        ],
        out_specs=[
            pl.BlockSpec((be, F), lambda i: (i, 0)),
            pl.BlockSpec((be, F), lambda i: (i, 0)),
        ],
        out_shape=[
            jax.ShapeDtypeStruct((e, F), jnp.float32),
            jax.ShapeDtypeStruct((e, F), jnp.float32),
        ],
    )(gr, gs, a_r, a_s, b0, w1, b1, w2, b2)


def _stats_body(sp_ref, qp_ref, cp_ref, xm_ref, xv_ref, bnd_ref):
    s = sp_ref[0] + sp_ref[1]
    q = qp_ref[0] + qp_ref[1]
    cnt = (cp_ref[0] + cp_ref[1])[:, :1]
    cc = jnp.clip(cnt, 1.0, None)
    xm = s / cc
    xv = jnp.maximum(q / cc - xm * xm, 0.0)
    xm_ref[...] = xm
    xv_ref[...] = xv
    bnd_ref[...] = xm + jnp.sqrt(xv * jnp.clip(cnt - 1.0, 0.0, None))


def _stats(sp, qp, cp):
    n = sp.shape[1]
    bn = 1000
    o = pl.BlockSpec((bn, F), lambda i: (i, 0))
    return pl.pallas_call(
        _stats_body,
        grid=(n // bn,),
        in_specs=[
            pl.BlockSpec((2, bn, F), lambda i: (0, i, 0)),
            pl.BlockSpec((2, bn, F), lambda i: (0, i, 0)),
            pl.BlockSpec((2, bn, 16), lambda i: (0, i, 0)),
        ],
        out_specs=[o, o, o],
        out_shape=[jax.ShapeDtypeStruct((n, F), jnp.float32)] * 3,
    )(sp, qp, cp)


def _expw_body(xs_ref, bd_ref, e1_ref, n1_ref, e10_ref, n10_ref):
    xs = xs_ref[...]
    u = xs - bd_ref[...]
    e1 = jnp.exp(u)
    e10 = jnp.exp(10.0 * u)
    e1_ref[...] = e1
    n1_ref[...] = xs * e1
    e10_ref[...] = e10
    n10_ref[...] = xs * e10


def _expw(xs, bndr):
    e = xs.shape[0]
    be = 3200
    sp = pl.BlockSpec((be, F), lambda i: (i, 0))
    return pl.pallas_call(
        _expw_body,
        grid=(e // be,),
        in_specs=[sp, sp],
        out_specs=[sp, sp, sp, sp],
        out_shape=[jax.ShapeDtypeStruct((e, F), jnp.float32)] * 4,
    )(xs, bndr)


def _final_body(xm_ref, xv_ref, d1_ref, n1_ref, d10_ref, n10_ref,
                w0_ref, b0_ref, w1_ref, b1_ref, w2_ref, b2_ref, out_ref):
    xm = xm_ref[...]
    xv = xv_ref[...]
    sm1 = (n1_ref[0] + n1_ref[1]) / jnp.clip(d1_ref[0] + d1_ref[1], MIN_NORM, None)
    sm10 = (n10_ref[0] + n10_ref[1]) / jnp.clip(d10_ref[0] + d10_ref[1], MIN_NORM, None)
    cat = jnp.concatenate([xm, xv, sm1, sm10], axis=-1)
    h = jax.nn.gelu(_mm(cat, w0_ref[...]) + b0_ref[...])
    h = jax.nn.gelu(_mm(h, w1_ref[...]) + b1_ref[...])
    xa = _mm(h, w2_ref[...]) + b2_ref[...]
    xa = _proj(_expmap0(xa))
    xt = jax.nn.relu(_logmap0(xa))
    out_ref[...] = _proj(_expmap0(xt))


def _final(xm, xv, d1p, n1p, d10p, n10p, w0, b0, w1, b1, w2, b2):
    n = xm.shape[0]
    bn = 1000
    h = w0.shape[0]
    nb = pl.BlockSpec((bn, F), lambda i: (i, 0))
    pb = pl.BlockSpec((2, bn, F), lambda i: (0, i, 0))
    full = lambda shp: pl.BlockSpec(shp, lambda i: (0, 0))
    return pl.pallas_call(
        _final_body,
        grid=(n // bn,),
        in_specs=[nb, nb, pb, pb, pb, pb,
                  full((h, 4 * F)), full((1, h)),
                  full((h, h)), full((1, h)),
                  full((F, h)), full((1, F))],
        out_specs=nb,
        out_shape=jax.ShapeDtypeStruct((n, F), jnp.float32),
    )(xm, xv, d1p, n1p, d10p, n10p, w0, b0, w1, b1, w2, b2)


_CHUNK = 80
_GCHUNK = 128
_KB = 5
_SKB = 3


def _sc_mesh():
    return plsc.VectorSubcoreMesh(core_axis_name="c", subcore_axis_name="s",
                                  num_cores=NC, num_subcores=NS)


def _sc_gather(table, idx):
    e = idx.shape[0]
    per_w = e // NW
    n_full = per_w // _GCHUNK
    tail = per_w - n_full * _GCHUNK
    groups = n_full // _KB
    rem = n_full - groups * _KB
    d = table.shape[1]

    @functools.partial(
        pl.kernel,
        mesh=_sc_mesh(),
        out_type=jax.ShapeDtypeStruct((e, d), jnp.float32),
        scratch_types=[
            pltpu.VMEM((per_w,), jnp.int32),
            pltpu.VMEM((_KB, _GCHUNK, d), jnp.float32),
            pltpu.SemaphoreType.DMA,
            pltpu.SemaphoreType.DMA,
        ],
    )
    def k(table_hbm, idx_hbm, out_hbm, idx_v, rows_v, gsem, ssem):
        c = lax.axis_index("c")
        s = lax.axis_index("s")
        wid = s * NC + c
        base = wid * per_w
        pltpu.sync_copy(idx_hbm.at[wid], idx_v)

        def do_chunk_sync(off, size, b):
            pltpu.async_copy(table_hbm.at[idx_v.at[pl.ds(off, size)]],
                             rows_v.at[b, pl.ds(0, size)], gsem).wait()
            pltpu.sync_copy(rows_v.at[b, pl.ds(0, size)],
                            out_hbm.at[pl.ds(base + off, size)])

        def group(g, carry):
            j0 = g * _KB
            gds = [
                pltpu.async_copy(
                    table_hbm.at[idx_v.at[pl.ds((j0 + b) * _GCHUNK, _GCHUNK)]],
                    rows_v.at[b], gsem)
                for b in range(_KB)
            ]
            for dsc in gds:
                dsc.wait()
            sds = [
                pltpu.async_copy(
                    rows_v.at[b],
                    out_hbm.at[pl.ds(base + (j0 + b) * _GCHUNK, _GCHUNK)],
                    ssem)
                for b in range(_KB)
            ]
            for dsc in sds:
                dsc.wait()
            return carry

        lax.fori_loop(0, groups, group, 0)
        for t in range(rem):
            do_chunk_sync((groups * _KB + t) * _GCHUNK, _GCHUNK, t)
        if tail:
            do_chunk_sync(n_full * _GCHUNK, tail, 0)

    return k(table, idx.reshape(NW, per_w))


def _sc_scatter_add(vals_list, idx, zeros, n, skb=_SKB):
    nv = len(vals_list)
    e, d = vals_list[0].shape
    per_w = e // NW
    n_chunks = per_w // _CHUNK
    groups = n_chunks // skb
    rem = n_chunks - groups * skb
    rows_t = (n // NS) // 8 * 8
    tail = n - NS * rows_t

    @functools.partial(
        pl.kernel,
        mesh=_sc_mesh(),
        out_type=[jax.ShapeDtypeStruct((NC, n, d), jnp.float32)] * nv,
        scratch_types=[
            pltpu.VMEM((n_chunks, _CHUNK), jnp.int32),
            pltpu.VMEM((skb, _CHUNK, d), jnp.float32),
            pltpu.VMEM_SHARED((n, d), jnp.float32),
            pltpu.SemaphoreType.DMA,
            pltpu.SemaphoreType.DMA,
        ],
    )
    def k(*refs):
        vin = refs[:nv]
        idx_hbm, zeros_hbm = refs[nv], refs[nv + 1]
        outs = refs[nv + 2:nv + 2 + nv]
        idx_v, vals_v, acc_sh, lsem, scsem = refs[nv + 2 + nv:]
        c = lax.axis_index("c")
        s = lax.axis_index("s")
        wid = s * NC + c
        base = wid * per_w
        pltpu.sync_copy(idx_hbm.at[wid], idx_v)

        def zero_acc():
            pltpu.sync_copy(zeros_hbm.at[pl.ds(s * rows_t, rows_t)],
                            acc_sh.at[pl.ds(s * rows_t, rows_t)])
            if tail:
                @pl.when(s == 0)
                def _():
                    pltpu.sync_copy(zeros_hbm.at[pl.ds(NS * rows_t, tail)],
                                    acc_sh.at[pl.ds(NS * rows_t, tail)])

        def export(out_hbm):
            pltpu.sync_copy(acc_sh.at[pl.ds(s * rows_t, rows_t)],
                            out_hbm.at[c, pl.ds(s * rows_t, rows_t)])
            if tail:
                @pl.when(s == 0)
                def _():
                    pltpu.sync_copy(acc_sh.at[pl.ds(NS * rows_t, tail)],
                                    out_hbm.at[c, pl.ds(NS * rows_t, tail)])

        for vals_hbm, out_hbm in zip(vin, outs):
            zero_acc()
            plsc.subcore_barrier()

            def group(g, carry, vals_hbm=vals_hbm):
                j0 = g * skb
                lds = [
                    pltpu.async_copy(
                        vals_hbm.at[pl.ds(base + (j0 + b) * _CHUNK, _CHUNK)],
                        vals_v.at[b], lsem)
                    for b in range(skb)
                ]
                for dsc in lds:
                    dsc.wait()
                sds = [
                    pltpu.async_copy(vals_v.at[b],
                                     acc_sh.at[idx_v.at[j0 + b]],
                                     scsem, add=True)
                    for b in range(skb)
                ]
                for dsc in sds:
                    dsc.wait()
                return carry

            lax.fori_loop(0, groups, group, 0)
            for t in range(rem):
                j = groups * skb + t
                pltpu.sync_copy(vals_hbm.at[pl.ds(base + j * _CHUNK, _CHUNK)],
                                vals_v.at[t])
                pltpu.sync_copy(vals_v.at[t], acc_sh.at[idx_v.at[j]], add=True)
            plsc.subcore_barrier()
            export(out_hbm)
            plsc.subcore_barrier()

    out = k(*vals_list, idx.reshape(NW, n_chunks, _CHUNK), zeros)
    return out if nv > 1 else [out]


def _sc_count(idx, ones_chunk, zeros, n):
    nw_, n_chunks, chunk = idx.shape
    d = ones_chunk.shape[1]
    kb = 8
    groups = n_chunks // kb
    rem = n_chunks - groups * kb
    rows_t = (n // NS) // 8 * 8
    tail = n - NS * rows_t

    @functools.partial(
        pl.kernel,
        mesh=_sc_mesh(),
        out_type=jax.ShapeDtypeStruct((NC, n, d), jnp.float32),
        scratch_types=[
            pltpu.VMEM((n_chunks, chunk), jnp.int32),
            pltpu.VMEM((chunk, d), jnp.float32),
            pltpu.VMEM_SHARED((n, d), jnp.float32),
            pltpu.SemaphoreType.DMA,
        ],
    )
    def k(idx_hbm, ones_hbm, zeros_hbm, out_hbm, idx_v, ones_v, acc_sh, scsem):
        c = lax.axis_index("c")
        s = lax.axis_index("s")
        wid = s * NC + c
        pltpu.sync_copy(zeros_hbm.at[pl.ds(s * rows_t, rows_t)],
                        acc_sh.at[pl.ds(s * rows_t, rows_t)])
        if tail:
            @pl.when(s == 0)
            def _():
                pltpu.sync_copy(zeros_hbm.at[pl.ds(NS * rows_t, tail)],
                                acc_sh.at[pl.ds(NS * rows_t, tail)])
        pltpu.sync_copy(ones_hbm, ones_v)
        pltpu.sync_copy(idx_hbm.at[wid], idx_v)
        plsc.subcore_barrier()

        def group(g, carry):
            j0 = g * kb
            sds = [
                pltpu.async_copy(ones_v, acc_sh.at[idx_v.at[j0 + b]],
                                 scsem, add=True)
                for b in range(kb)
            ]
            for dsc in sds:
                dsc.wait()
            return carry

        lax.fori_loop(0, groups, group, 0)
        for t in range(rem):
            pltpu.sync_copy(ones_v, acc_sh.at[idx_v.at[groups * kb + t]],
                            add=True)
        plsc.subcore_barrier()
        pltpu.sync_copy(acc_sh.at[pl.ds(s * rows_t, rows_t)],
                        out_hbm.at[c, pl.ds(s * rows_t, rows_t)])
        if tail:
            @pl.when(s == 0)
            def _():
                pltpu.sync_copy(acc_sh.at[pl.ds(NS * rows_t, tail)],
                                out_hbm.at[c, pl.ds(NS * rows_t, tail)])

    return k(idx, ones_chunk, zeros)


def kernel(x, adj, key, W, b, ec_W0, ec_b0, ec_W1, ec_b1, ec_W2, ec_b2,
           ag_W0, ag_b0, ag_W1, ag_b1, ag_W2, ag_b2):
    n = x.shape[0]
    s, r = adj[0], adj[1]

    bias = b.reshape(1, -1)
    bn = jnp.clip(jnp.sqrt(jnp.sum(bias * bias, -1, keepdims=True)), MIN_NORM, None)
    eb = jnp.tanh(bn) * bias / bn
    ebn = jnp.clip(jnp.sqrt(jnp.sum(eb * eb, -1, keepdims=True)), MIN_NORM, None)
    maxnorm = 1.0 - 4e-3
    hyp_bias = jnp.where(ebn > maxnorm, eb / ebn * maxnorm, eb)
    a_r = ec_W0[:, :F] - ec_W0[:, 2 * F:]
    a_s = ec_W0[:, F:2 * F] + ec_W0[:, 2 * F:]
    row = lambda v: v.reshape(1, -1)

    e_num = adj.shape[1]
    ht = _node_prep(x, W, hyp_bias)
    gcat = _sc_gather(ht, jnp.concatenate([s, r]))
    ghts, ghtr = gcat[:e_num], gcat[e_num:]
    xs, xsq = _edge_mlp(ghtr, ghts, a_r, a_s, row(ec_b0), ec_W1, row(ec_b1),
                        ec_W2, row(ec_b2))

    z128 = jnp.zeros((n, F), jnp.float32)
    z16 = jnp.zeros((n, 16), jnp.float32)
    ones16 = jnp.ones((_CHUNK, 16), jnp.float32)
    cp = _sc_count(r.reshape(NW, -1, _CHUNK), ones16, z16, n)
    sp, qp = _sc_scatter_add([xs, xsq], r, z128, n)
    xm, xv, bnd = _stats(sp, qp, cp)

    bndr = _sc_gather(bnd, r)
    e1, n1, e10, n10 = _expw(xs, bndr)
    d1p, n1p, d10p, n10p = _sc_scatter_add([e1, n1, e10, n10], r, z128, n)

    out = _final(xm, xv, d1p, n1p, d10p, n10p,
                 ag_W0, row(ag_b0), ag_W1, row(ag_b1), ag_W2, row(ag_b2))
    return (out, adj)


# speedup vs baseline: 4.0448x; 1.0019x over previous
import functools

import jax
import jax.numpy as jnp
from jax import lax
from jax.experimental import pallas as pl
from jax.experimental.pallas import tpu as pltpu
from jax.experimental.pallas import tpu_sc as plsc

F = 128
MIN_NORM = 1e-15

NC = 2
NS = 16
NW = NC * NS


def _rnorm(x):
    return jnp.clip(jnp.sqrt(jnp.sum(x * x, axis=-1, keepdims=True)), MIN_NORM, None)


def _artanh(z):
    z = jnp.clip(z, -1.0 + 1e-7, 1.0 - 1e-7)
    return 0.5 * jnp.log((1.0 + z) / (1.0 - z))


def _proj(x):
    maxnorm = 1.0 - 4e-3
    n = _rnorm(x)
    return jnp.where(n > maxnorm, x / n * maxnorm, x)


def _expmap0(u):
    un = _rnorm(u)
    return jnp.tanh(un) * u / un


def _logmap0(p):
    pn = _rnorm(p)
    return _artanh(pn) * p / pn


def _mm(a, b_t):
    return lax.dot_general(a, b_t, (((1,), (1,)), ((), ())),
                           preferred_element_type=jnp.float32)


def _node_prep_body(x_ref, w_ref, hb_ref, ht_ref):
    x = x_ref[...]
    w = w_ref[...]
    hb = hb_ref[...]
    xn = _rnorm(x)
    mx = _mm(x, w)
    mxn = _rnorm(mx)
    res = jnp.tanh(mxn / xn * _artanh(xn)) * mx / mxn
    res = _proj(res)
    x2 = jnp.sum(res * res, -1, keepdims=True)
    y2 = jnp.sum(hb * hb, -1, keepdims=True)
    xy = jnp.sum(res * hb, -1, keepdims=True)
    num = (1.0 + 2.0 * xy + y2) * res + (1.0 - x2) * hb
    den = 1.0 + 2.0 * xy + x2 * y2
    h = _proj(num / jnp.clip(den, MIN_NORM, None))
    ht_ref[...] = _logmap0(h)


def _node_prep(x, w, hyp_bias):
    n = x.shape[0]
    bn = 1000
    return pl.pallas_call(
        _node_prep_body,
        grid=(n // bn,),
        in_specs=[
            pl.BlockSpec((bn, F), lambda i: (i, 0)),
            pl.BlockSpec((F, F), lambda i: (0, 0)),
            pl.BlockSpec((1, F), lambda i: (0, 0)),
        ],
        out_specs=pl.BlockSpec((bn, F), lambda i: (i, 0)),
        out_shape=jax.ShapeDtypeStruct((n, F), jnp.float32),
    )(x, w, hyp_bias)


def _edge_mlp_body(gr_ref, gs_ref, ar_ref, as_ref, b0_ref, w1_ref, b1_ref,
                   w2_ref, b2_ref, xs1_ref, xsq_ref):
    gr = gr_ref[...]
    gs = gs_ref[...]
    h1 = jax.nn.gelu(_mm(gr, ar_ref[...]) + _mm(gs, as_ref[...]) + b0_ref[...])
    h2 = jax.nn.gelu(_mm(h1, w1_ref[...]) + b1_ref[...])
    xs = _mm(h2, w2_ref[...]) + b2_ref[...]
    xs1_ref[...] = xs
    xsq_ref[...] = xs * xs


def _edge_mlp(gr, gs, a_r, a_s, b0, w1, b1, w2, b2):
    e = gr.shape[0]
    be = 3200
    h = a_r.shape[0]
    full = lambda shp: pl.BlockSpec(shp, lambda i: (0, 0))
    return pl.pallas_call(
        _edge_mlp_body,
        grid=(e // be,),
        in_specs=[
            pl.BlockSpec((be, F), lambda i: (i, 0)),
            pl.BlockSpec((be, F), lambda i: (i, 0)),
            full((h, F)), full((h, F)), full((1, h)),
            full((h, h)), full((1, h)),
            full((F, h)), full((1, F)),
        ],
        out_specs=[
            pl.BlockSpec((be, F), lambda i: (i, 0)),
            pl.BlockSpec((be, F), lambda i: (i, 0)),
        ],
        out_shape=[
            jax.ShapeDtypeStruct((e, F), jnp.float32),
            jax.ShapeDtypeStruct((e, F), jnp.float32),
        ],
    )(gr, gs, a_r, a_s, b0, w1, b1, w2, b2)


def _stats_body(sp_ref, qp_ref, cp_ref, xm_ref, xv_ref, bnd_ref):
    s = sp_ref[0] + sp_ref[1]
    q = qp_ref[0] + qp_ref[1]
    cnt = (cp_ref[0] + cp_ref[1])[:, :1]
    cc = jnp.clip(cnt, 1.0, None)
    xm = s / cc
    xv = jnp.maximum(q / cc - xm * xm, 0.0)
    xm_ref[...] = xm
    xv_ref[...] = xv
    bnd_ref[...] = xm + jnp.sqrt(xv * jnp.clip(cnt - 1.0, 0.0, None))


def _stats(sp, qp, cp):
    n = sp.shape[1]
    bn = 1000
    o = pl.BlockSpec((bn, F), lambda i: (i, 0))
    return pl.pallas_call(
        _stats_body,
        grid=(n // bn,),
        in_specs=[
            pl.BlockSpec((2, bn, F), lambda i: (0, i, 0)),
            pl.BlockSpec((2, bn, F), lambda i: (0, i, 0)),
            pl.BlockSpec((2, bn, 16), lambda i: (0, i, 0)),
        ],
        out_specs=[o, o, o],
        out_shape=[jax.ShapeDtypeStruct((n, F), jnp.float32)] * 3,
    )(sp, qp, cp)


def _expw_body(xs_ref, bd_ref, e1_ref, n1_ref, e10_ref, n10_ref):
    xs = xs_ref[...]
    u = xs - bd_ref[...]
    e1 = jnp.exp(u)
    e10 = jnp.exp(10.0 * u)
    e1_ref[...] = e1
    n1_ref[...] = xs * e1
    e10_ref[...] = e10
    n10_ref[...] = xs * e10


def _expw(xs, bndr):
    e = xs.shape[0]
    be = 3200
    sp = pl.BlockSpec((be, F), lambda i: (i, 0))
    return pl.pallas_call(
        _expw_body,
        grid=(e // be,),
        in_specs=[sp, sp],
        out_specs=[sp, sp, sp, sp],
        out_shape=[jax.ShapeDtypeStruct((e, F), jnp.float32)] * 4,
    )(xs, bndr)


def _final_body(xm_ref, xv_ref, d1_ref, n1_ref, d10_ref, n10_ref,
                w0_ref, b0_ref, w1_ref, b1_ref, w2_ref, b2_ref, out_ref):
    xm = xm_ref[...]
    xv = xv_ref[...]
    sm1 = (n1_ref[0] + n1_ref[1]) / jnp.clip(d1_ref[0] + d1_ref[1], MIN_NORM, None)
    sm10 = (n10_ref[0] + n10_ref[1]) / jnp.clip(d10_ref[0] + d10_ref[1], MIN_NORM, None)
    cat = jnp.concatenate([xm, xv, sm1, sm10], axis=-1)
    h = jax.nn.gelu(_mm(cat, w0_ref[...]) + b0_ref[...])
    h = jax.nn.gelu(_mm(h, w1_ref[...]) + b1_ref[...])
    xa = _mm(h, w2_ref[...]) + b2_ref[...]
    xa = _proj(_expmap0(xa))
    xt = jax.nn.relu(_logmap0(xa))
    out_ref[...] = _proj(_expmap0(xt))


def _final(xm, xv, d1p, n1p, d10p, n10p, w0, b0, w1, b1, w2, b2):
    n = xm.shape[0]
    bn = 1000
    h = w0.shape[0]
    nb = pl.BlockSpec((bn, F), lambda i: (i, 0))
    pb = pl.BlockSpec((2, bn, F), lambda i: (0, i, 0))
    full = lambda shp: pl.BlockSpec(shp, lambda i: (0, 0))
    return pl.pallas_call(
        _final_body,
        grid=(n // bn,),
        in_specs=[nb, nb, pb, pb, pb, pb,
                  full((h, 4 * F)), full((1, h)),
                  full((h, h)), full((1, h)),
                  full((F, h)), full((1, F))],
        out_specs=nb,
        out_shape=jax.ShapeDtypeStruct((n, F), jnp.float32),
    )(xm, xv, d1p, n1p, d10p, n10p, w0, b0, w1, b1, w2, b2)


_CHUNK = 80
_GCHUNK = 128
_KB = 6
_SKB = 3


def _sc_mesh():
    return plsc.VectorSubcoreMesh(core_axis_name="c", subcore_axis_name="s",
                                  num_cores=NC, num_subcores=NS)


def _sc_gather(table, idx):
    e = idx.shape[0]
    per_w = e // NW
    n_full = per_w // _GCHUNK
    tail = per_w - n_full * _GCHUNK
    groups = n_full // _KB
    rem = n_full - groups * _KB
    d = table.shape[1]

    @functools.partial(
        pl.kernel,
        mesh=_sc_mesh(),
        out_type=jax.ShapeDtypeStruct((e, d), jnp.float32),
        scratch_types=[
            pltpu.VMEM((per_w,), jnp.int32),
            pltpu.VMEM((_KB, _GCHUNK, d), jnp.float32),
            pltpu.SemaphoreType.DMA,
            pltpu.SemaphoreType.DMA,
        ],
    )
    def k(table_hbm, idx_hbm, out_hbm, idx_v, rows_v, gsem, ssem):
        c = lax.axis_index("c")
        s = lax.axis_index("s")
        wid = s * NC + c
        base = wid * per_w
        pltpu.sync_copy(idx_hbm.at[wid], idx_v)

        def do_chunk_sync(off, size, b):
            pltpu.async_copy(table_hbm.at[idx_v.at[pl.ds(off, size)]],
                             rows_v.at[b, pl.ds(0, size)], gsem).wait()
            pltpu.sync_copy(rows_v.at[b, pl.ds(0, size)],
                            out_hbm.at[pl.ds(base + off, size)])

        def group(g, carry):
            j0 = g * _KB
            gds = [
                pltpu.async_copy(
                    table_hbm.at[idx_v.at[pl.ds((j0 + b) * _GCHUNK, _GCHUNK)]],
                    rows_v.at[b], gsem)
                for b in range(_KB)
            ]
            for dsc in gds:
                dsc.wait()
            sds = [
                pltpu.async_copy(
                    rows_v.at[b],
                    out_hbm.at[pl.ds(base + (j0 + b) * _GCHUNK, _GCHUNK)],
                    ssem)
                for b in range(_KB)
            ]
            for dsc in sds:
                dsc.wait()
            return carry

        lax.fori_loop(0, groups, group, 0)
        for t in range(rem):
            do_chunk_sync((groups * _KB + t) * _GCHUNK, _GCHUNK, t)
        if tail:
            do_chunk_sync(n_full * _GCHUNK, tail, 0)

    return k(table, idx.reshape(NW, per_w))


def _sc_scatter_add(vals_list, idx, zeros, n, skb=_SKB):
    nv = len(vals_list)
    e, d = vals_list[0].shape
    per_w = e // NW
    n_chunks = per_w // _CHUNK
    groups = n_chunks // skb
    rem = n_chunks - groups * skb
    rows_t = (n // NS) // 8 * 8
    tail = n - NS * rows_t

    @functools.partial(
        pl.kernel,
        mesh=_sc_mesh(),
        out_type=[jax.ShapeDtypeStruct((NC, n, d), jnp.float32)] * nv,
        scratch_types=[
            pltpu.VMEM((n_chunks, _CHUNK), jnp.int32),
            pltpu.VMEM((skb, _CHUNK, d), jnp.float32),
            pltpu.VMEM_SHARED((n, d), jnp.float32),
            pltpu.SemaphoreType.DMA,
            pltpu.SemaphoreType.DMA,
        ],
    )
    def k(*refs):
        vin = refs[:nv]
        idx_hbm, zeros_hbm = refs[nv], refs[nv + 1]
        outs = refs[nv + 2:nv + 2 + nv]
        idx_v, vals_v, acc_sh, lsem, scsem = refs[nv + 2 + nv:]
        c = lax.axis_index("c")
        s = lax.axis_index("s")
        wid = s * NC + c
        base = wid * per_w
        pltpu.sync_copy(idx_hbm.at[wid], idx_v)

        def zero_acc():
            pltpu.sync_copy(zeros_hbm.at[pl.ds(s * rows_t, rows_t)],
                            acc_sh.at[pl.ds(s * rows_t, rows_t)])
            if tail:
                @pl.when(s == 0)
                def _():
                    pltpu.sync_copy(zeros_hbm.at[pl.ds(NS * rows_t, tail)],
                                    acc_sh.at[pl.ds(NS * rows_t, tail)])

        def export(out_hbm):
            pltpu.sync_copy(acc_sh.at[pl.ds(s * rows_t, rows_t)],
                            out_hbm.at[c, pl.ds(s * rows_t, rows_t)])
            if tail:
                @pl.when(s == 0)
                def _():
                    pltpu.sync_copy(acc_sh.at[pl.ds(NS * rows_t, tail)],
                                    out_hbm.at[c, pl.ds(NS * rows_t, tail)])

        for vals_hbm, out_hbm in zip(vin, outs):
            zero_acc()
            plsc.subcore_barrier()

            def group(g, carry, vals_hbm=vals_hbm):
                j0 = g * skb
                lds = [
                    pltpu.async_copy(
                        vals_hbm.at[pl.ds(base + (j0 + b) * _CHUNK, _CHUNK)],
                        vals_v.at[b], lsem)
                    for b in range(skb)
                ]
                for dsc in lds:
                    dsc.wait()
                sds = [
                    pltpu.async_copy(vals_v.at[b],
                                     acc_sh.at[idx_v.at[j0 + b]],
                                     scsem, add=True)
                    for b in range(skb)
                ]
                for dsc in sds:
                    dsc.wait()
                return carry

            lax.fori_loop(0, groups, group, 0)
            for t in range(rem):
                j = groups * skb + t
                pltpu.sync_copy(vals_hbm.at[pl.ds(base + j * _CHUNK, _CHUNK)],
                                vals_v.at[t])
                pltpu.sync_copy(vals_v.at[t], acc_sh.at[idx_v.at[j]], add=True)
            plsc.subcore_barrier()
            export(out_hbm)
            plsc.subcore_barrier()

    out = k(*vals_list, idx.reshape(NW, n_chunks, _CHUNK), zeros)
    return out if nv > 1 else [out]


def _sc_count(idx, ones_chunk, zeros, n):
    nw_, n_chunks, chunk = idx.shape
    d = ones_chunk.shape[1]
    kb = 8
    groups = n_chunks // kb
    rem = n_chunks - groups * kb
    rows_t = (n // NS) // 8 * 8
    tail = n - NS * rows_t

    @functools.partial(
        pl.kernel,
        mesh=_sc_mesh(),
        out_type=jax.ShapeDtypeStruct((NC, n, d), jnp.float32),
        scratch_types=[
            pltpu.VMEM((n_chunks, chunk), jnp.int32),
            pltpu.VMEM((chunk, d), jnp.float32),
            pltpu.VMEM_SHARED((n, d), jnp.float32),
            pltpu.SemaphoreType.DMA,
        ],
    )
    def k(idx_hbm, ones_hbm, zeros_hbm, out_hbm, idx_v, ones_v, acc_sh, scsem):
        c = lax.axis_index("c")
        s = lax.axis_index("s")
        wid = s * NC + c
        pltpu.sync_copy(zeros_hbm.at[pl.ds(s * rows_t, rows_t)],
                        acc_sh.at[pl.ds(s * rows_t, rows_t)])
        if tail:
            @pl.when(s == 0)
            def _():
                pltpu.sync_copy(zeros_hbm.at[pl.ds(NS * rows_t, tail)],
                                acc_sh.at[pl.ds(NS * rows_t, tail)])
        pltpu.sync_copy(ones_hbm, ones_v)
        pltpu.sync_copy(idx_hbm.at[wid], idx_v)
        plsc.subcore_barrier()

        def group(g, carry):
            j0 = g * kb
            sds = [
                pltpu.async_copy(ones_v, acc_sh.at[idx_v.at[j0 + b]],
                                 scsem, add=True)
                for b in range(kb)
            ]
            for dsc in sds:
                dsc.wait()
            return carry

        lax.fori_loop(0, groups, group, 0)
        for t in range(rem):
            pltpu.sync_copy(ones_v, acc_sh.at[idx_v.at[groups * kb + t]],
                            add=True)
        plsc.subcore_barrier()
        pltpu.sync_copy(acc_sh.at[pl.ds(s * rows_t, rows_t)],
                        out_hbm.at[c, pl.ds(s * rows_t, rows_t)])
        if tail:
            @pl.when(s == 0)
            def _():
                pltpu.sync_copy(acc_sh.at[pl.ds(NS * rows_t, tail)],
                                out_hbm.at[c, pl.ds(NS * rows_t, tail)])

    return k(idx, ones_chunk, zeros)


def kernel(x, adj, key, W, b, ec_W0, ec_b0, ec_W1, ec_b1, ec_W2, ec_b2,
           ag_W0, ag_b0, ag_W1, ag_b1, ag_W2, ag_b2):
    n = x.shape[0]
    s, r = adj[0], adj[1]

    bias = b.reshape(1, -1)
    bn = jnp.clip(jnp.sqrt(jnp.sum(bias * bias, -1, keepdims=True)), MIN_NORM, None)
    eb = jnp.tanh(bn) * bias / bn
    ebn = jnp.clip(jnp.sqrt(jnp.sum(eb * eb, -1, keepdims=True)), MIN_NORM, None)
    maxnorm = 1.0 - 4e-3
    hyp_bias = jnp.where(ebn > maxnorm, eb / ebn * maxnorm, eb)
    a_r = ec_W0[:, :F] - ec_W0[:, 2 * F:]
    a_s = ec_W0[:, F:2 * F] + ec_W0[:, 2 * F:]
    row = lambda v: v.reshape(1, -1)

    e_num = adj.shape[1]
    ht = _node_prep(x, W, hyp_bias)
    gcat = _sc_gather(ht, jnp.concatenate([s, r]))
    ghts, ghtr = gcat[:e_num], gcat[e_num:]
    xs, xsq = _edge_mlp(ghtr, ghts, a_r, a_s, row(ec_b0), ec_W1, row(ec_b1),
                        ec_W2, row(ec_b2))

    z128 = jnp.zeros((n, F), jnp.float32)
    z16 = jnp.zeros((n, 16), jnp.float32)
    ones16 = jnp.ones((_CHUNK, 16), jnp.float32)
    cp = _sc_count(r.reshape(NW, -1, _CHUNK), ones16, z16, n)
    sp, qp = _sc_scatter_add([xs, xsq], r, z128, n)
    xm, xv, bnd = _stats(sp, qp, cp)

    bndr = _sc_gather(bnd, r)
    e1, n1, e10, n10 = _expw(xs, bndr)
    d1p, n1p, d10p, n10p = _sc_scatter_add([e1, n1, e10, n10], r, z128, n)

    out = _final(xm, xv, d1p, n1p, d10p, n10p,
                 ag_W0, row(ag_b0), ag_W1, row(ag_b1), ag_W2, row(ag_b2))
    return (out, adj)
